# Initial kernel scaffold; baseline (speedup 1.0000x reference)
import jax
import jax.numpy as jnp
from jax.experimental import pallas as pl


def kernel(emb, edge_index, rbf, i, j, W_gat, att_src, att_dst, b_gat, W_rbf, b_rbf, W_lin, b_lin, W_out, b_out):
    raise NotImplementedError("write your pallas kernel here")


# speedup vs baseline: 12.6670x; 12.6670x over previous
import jax
import jax.numpy as jnp
from jax import lax
from jax.experimental import pallas as pl
from jax.experimental.pallas import tpu as pltpu
from jax.experimental.pallas import tpu_sc as plsc

N = 10000
E = 320000
HID = 128
HEAD = 4
RAD = 16
BOND = 128

NC = 2
NS = 16
NW = NC * NS
ET = E // NW
RT = N // NS

_f32 = jnp.float32
_i32 = jnp.int32


BN1 = 400
G1 = N // BN1


def _k1_body(emb_ref, wg_ref, asv_ref, adv_ref,
             xw0_ref, xw1_ref, xw2_ref, xw3_ref, as_ref, ad_ref):
    xw = jnp.dot(emb_ref[...], wg_ref[...], preferred_element_type=_f32)
    asv = asv_ref[...]
    adv = adv_ref[...]
    xw_refs = (xw0_ref, xw1_ref, xw2_ref, xw3_ref)
    a_s = []
    a_d = []
    for h in range(HEAD):
        xh = xw[:, h * HID:(h + 1) * HID]
        xw_refs[h][...] = xh
        a_s.append(jnp.sum(xh * asv[h][None, :], axis=1)[:, None])
        a_d.append(jnp.sum(xh * adv[h][None, :], axis=1)[:, None])
    as_ref[...] = jnp.concatenate(a_s, axis=1)
    ad_ref[...] = jnp.concatenate(a_d, axis=1)


def _k1(emb, W_gat, att_src, att_dst):
    xw_spec = pl.BlockSpec((BN1, HID), lambda i: (i, 0))
    xw_shape = jax.ShapeDtypeStruct((N, HID), _f32)
    a_spec = pl.BlockSpec((BN1, HEAD), lambda i: (i, 0))
    a_shape = jax.ShapeDtypeStruct((N, HEAD), _f32)
    return pl.pallas_call(
        _k1_body,
        grid=(G1,),
        in_specs=[
            pl.BlockSpec((BN1, HID), lambda i: (i, 0)),
            pl.BlockSpec((HID, HEAD * HID), lambda i: (0, 0)),
            pl.BlockSpec((HEAD, HID), lambda i: (0, 0)),
            pl.BlockSpec((HEAD, HID), lambda i: (0, 0)),
        ],
        out_specs=[xw_spec, xw_spec, xw_spec, xw_spec, a_spec, a_spec],
        out_shape=[xw_shape, xw_shape, xw_shape, xw_shape, a_shape, a_shape],
    )(emb, W_gat, att_src, att_dst)


C1 = 80
NCH1 = ET // C1
DRW = 320


def _s1_body(as_hbm, ad_hbm, src_hbm, dst_hbm, denp_hbm, ex_hbm,
             as_v, ad_v, acc_v, srcv, dstv, exb):
    c = lax.axis_index("c")
    s = lax.axis_index("s")
    wid = c * NS + s
    base = wid * ET
    pltpu.sync_copy(as_hbm, as_v)
    pltpu.sync_copy(ad_hbm, ad_v)

    @pl.loop(0, DRW)
    def _zero(z):
        for g in range(8):
            acc_v[z, pl.ds(g * 16, 16)] = jnp.zeros((16,), _f32)

    @pl.loop(0, NCH1)
    def _chunk(ch):
        off = base + ch * C1
        pltpu.sync_copy(src_hbm.at[pl.ds(off, C1)], srcv)
        pltpu.sync_copy(dst_hbm.at[pl.ds(off, C1)], dstv)
        for g in range(C1 // 16):
            sv = srcv[pl.ds(g * 16, 16)]
            dv = dstv[pl.ds(g * 16, 16)]
            for h in range(HEAD):
                a = plsc.load_gather(as_v, [sv * HEAD + h])
                b = plsc.load_gather(ad_v, [dv * HEAD + h])
                al = a + b
                ex = jnp.exp(jnp.maximum(al, 0.2 * al))
                exb[h, pl.ds(g * 16, 16)] = ex
                fidx = dv * HEAD + h
                plsc.addupdate_scatter(acc_v, [fidx >> 7, fidx & 127], ex)
        for h in range(HEAD):
            pltpu.sync_copy(exb.at[h], ex_hbm.at[pl.ds(h * E + off, C1)])

    pltpu.sync_copy(acc_v, denp_hbm.at[wid])


def _s1(asT, adT, src, dst):
    mesh = plsc.VectorSubcoreMesh(core_axis_name="c", subcore_axis_name="s",
                                  num_cores=NC, num_subcores=NS)
    return pl.kernel(
        _s1_body,
        out_type=[jax.ShapeDtypeStruct((NW, DRW, 128), _f32),
                  jax.ShapeDtypeStruct((HEAD * E,), _f32)],
        mesh=mesh,
        compiler_params=pltpu.CompilerParams(needs_layout_passes=False),
        scratch_types=[
            pltpu.VMEM((N * HEAD,), _f32),
            pltpu.VMEM((N * HEAD,), _f32),
            pltpu.VMEM((DRW, 128), _f32),
            pltpu.VMEM((C1,), _i32),
            pltpu.VMEM((C1,), _i32),
            pltpu.VMEM((HEAD, C1), _f32),
        ],
    )(asT, adT, src, dst)


C2 = 80
NCH2 = ET // C2
SL = 624
TAIL = N - NS * SL


def _s2_body(xw_hbm, ex_hbm, src_hbm, dst_hbm, sp_hbm,
             rows_v, zbuf, srcv, dstv, idxv, exb, acc_sh, sem):
    c = lax.axis_index("c")
    s = lax.axis_index("s")
    base = (c * NS + s) * ET

    @pl.loop(0, TAIL)
    def _zb(z):
        for g in range(HID // 16):
            zbuf[z, pl.ds(g * 16, 16)] = jnp.zeros((16,), _f32)

    @pl.loop(0, HEAD)
    def _head(h):
        @pl.loop(0, SL // TAIL)
        def _zslice(z):
            pltpu.sync_copy(zbuf, acc_sh.at[pl.ds(s * SL + z * TAIL, TAIL)])

        @pl.when(s == 0)
        def _ztail():
            pltpu.sync_copy(zbuf, acc_sh.at[pl.ds(NS * SL, TAIL)])
        plsc.subcore_barrier()

        @pl.loop(0, NCH2)
        def _chunk(ch):
            off = base + ch * C2
            pltpu.sync_copy(src_hbm.at[pl.ds(off, C2)], srcv)
            pltpu.sync_copy(ex_hbm.at[pl.ds(h * E + off, C2)], exb)
            pltpu.sync_copy(dst_hbm.at[pl.ds(off, C2)], dstv)
            hN = h * N
            for g in range(C2 // 16):
                idxv[pl.ds(g * 16, 16)] = srcv[pl.ds(g * 16, 16)] + hN
            pltpu.async_copy(xw_hbm.at[idxv], rows_v, sem).wait()
            for gk in range(C2 // 16):
                mv = exb[pl.ds(gk * 16, 16)]
                for kk in range(16):
                    k = gk * 16 + kk
                    m = mv[kk]
                    for g in range(HID // 16):
                        sl = pl.ds(g * 16, 16)
                        rows_v[k, sl] = rows_v[k, sl] * m
            pltpu.sync_copy(rows_v, acc_sh.at[dstv], add=True)

        plsc.subcore_barrier()
        pltpu.sync_copy(acc_sh.at[pl.ds(s * SL, SL)],
                        sp_hbm.at[c, h, pl.ds(s * SL, SL)])

        @pl.when(s == 0)
        def _ctail():
            pltpu.sync_copy(acc_sh.at[pl.ds(NS * SL, TAIL)],
                            sp_hbm.at[c, h, pl.ds(NS * SL, TAIL)])
        plsc.subcore_barrier()


def _s2(xwall, exmat, src, dst):
    mesh = plsc.VectorSubcoreMesh(core_axis_name="c", subcore_axis_name="s",
                                  num_cores=NC, num_subcores=NS)
    return pl.kernel(
        _s2_body,
        out_type=jax.ShapeDtypeStruct((NC, HEAD, N, HID), _f32),
        mesh=mesh,
        compiler_params=pltpu.CompilerParams(needs_layout_passes=False),
        scratch_types=[
            pltpu.VMEM((C2, HID), _f32),
            pltpu.VMEM((TAIL, HID), _f32),
            pltpu.VMEM((C2,), _i32),
            pltpu.VMEM((C2,), _i32),
            pltpu.VMEM((C2,), _i32),
            pltpu.VMEM((C2,), _f32),
            pltpu.VMEM_SHARED((N, HID), _f32),
            pltpu.SemaphoreType.DMA,
        ],
    )(xwall, exmat, src, dst)


BN2 = 400
G2 = N // BN2


def _k2_body(sp_ref, dn_ref, emb_ref, bgat_ref, wlin_ref, blin_ref,
             w1_ref, w2_ref, x_ref, u_ref, v_ref):
    sp = sp_ref[...]
    den = jnp.sum(dn_ref[...], axis=0)
    emb = emb_ref[...]
    bgat = bgat_ref[...]
    yns = []
    for h in range(HEAD):
        sh = sp[0, h] + sp[1, h]
        agg = sh / (den[:, h:h + 1] + 1e-16)
        y = agg + bgat[h][None, :] + emb
        m = jnp.mean(y, axis=1, keepdims=True)
        yc = y - m
        var = jnp.mean(yc * yc, axis=1, keepdims=True)
        yns.append(yc * lax.rsqrt(var + 1e-5))
    xcat = jnp.concatenate(yns, axis=1)
    xx = jnp.dot(xcat, wlin_ref[...], preferred_element_type=_f32)
    xx = jnp.maximum(xx + blin_ref[...], 0.0)
    x_ref[...] = xx
    u_ref[...] = jnp.dot(xx, w1_ref[...], preferred_element_type=_f32)
    v_ref[...] = jnp.dot(xx, w2_ref[...], preferred_element_type=_f32)


def _k2(sp, denp3, emb, b_gat4, W_lin, b_lin2, W1, W2):
    return pl.pallas_call(
        _k2_body,
        grid=(G2,),
        in_specs=[
            pl.BlockSpec((NC, HEAD, BN2, HID), lambda i: (0, 0, i, 0)),
            pl.BlockSpec((NW, BN2, HEAD), lambda i: (0, i, 0)),
            pl.BlockSpec((BN2, HID), lambda i: (i, 0)),
            pl.BlockSpec((HEAD, HID), lambda i: (0, 0)),
            pl.BlockSpec((HEAD * HID, HID), lambda i: (0, 0)),
            pl.BlockSpec((1, HID), lambda i: (0, 0)),
            pl.BlockSpec((HID, HID), lambda i: (0, 0)),
            pl.BlockSpec((HID, HID), lambda i: (0, 0)),
        ],
        out_specs=[
            pl.BlockSpec((BN2, HID), lambda i: (i, 0)),
            pl.BlockSpec((BN2, HID), lambda i: (i, 0)),
            pl.BlockSpec((BN2, HID), lambda i: (i, 0)),
        ],
        out_shape=[
            jax.ShapeDtypeStruct((N, HID), _f32),
            jax.ShapeDtypeStruct((N, HID), _f32),
            jax.ShapeDtypeStruct((N, HID), _f32),
        ],
    )(sp, denp3, emb, b_gat4, W_lin, b_lin2, W1, W2)


BE3 = 2000
G3 = E // BE3


def _k3_body(rbf_ref, wr_ref, br_ref, w3_ref, bo_ref, t_ref):
    h1 = jnp.dot(rbf_ref[...], wr_ref[...], preferred_element_type=_f32)
    h1 = jnp.maximum(h1 + br_ref[...], 0.0)
    t_ref[...] = jnp.dot(h1, w3_ref[...], preferred_element_type=_f32) + bo_ref[...]


def _k3(rbf, W_rbf, b_rbf2, W3, b_out2):
    return pl.pallas_call(
        _k3_body,
        grid=(G3,),
        in_specs=[
            pl.BlockSpec((BE3, RAD), lambda i: (i, 0)),
            pl.BlockSpec((RAD, HID), lambda i: (0, 0)),
            pl.BlockSpec((1, HID), lambda i: (0, 0)),
            pl.BlockSpec((HID, BOND), lambda i: (0, 0)),
            pl.BlockSpec((1, BOND), lambda i: (0, 0)),
        ],
        out_specs=[pl.BlockSpec((BE3, BOND), lambda i: (i, 0))],
        out_shape=[jax.ShapeDtypeStruct((E, BOND), _f32)],
    )(rbf, W_rbf, b_rbf2, W3, b_out2)[0]


C3 = 40
NCH3 = ET // C3


def _s3_body(u_hbm, v_hbm, t_hbm, i_hbm, j_hbm, out_hbm,
             iv, jv, bu, bv, bt, sem):
    c = lax.axis_index("c")
    s = lax.axis_index("s")
    base = (c * NS + s) * ET

    @pl.loop(0, NCH3)
    def _chunk(ch):
        off = base + ch * C3
        pltpu.sync_copy(i_hbm.at[pl.ds(off, C3)], iv)
        pltpu.sync_copy(j_hbm.at[pl.ds(off, C3)], jv)
        cu = pltpu.async_copy(u_hbm.at[iv], bu, sem)
        cv = pltpu.async_copy(v_hbm.at[jv], bv, sem)
        pltpu.sync_copy(t_hbm.at[pl.ds(off, C3)], bt)
        cu.wait()
        cv.wait()
        for k in range(C3):
            for g in range(BOND // 16):
                sl = pl.ds(g * 16, 16)
                bu[k, sl] = jnp.maximum(bu[k, sl] + bv[k, sl] + bt[k, sl], 0.0)
        pltpu.sync_copy(bu, out_hbm.at[pl.ds(off, C3)])


def _s3(u, v, t, i, j):
    mesh = plsc.VectorSubcoreMesh(core_axis_name="c", subcore_axis_name="s",
                                  num_cores=NC, num_subcores=NS)
    return pl.kernel(
        _s3_body,
        out_type=jax.ShapeDtypeStruct((E, BOND), _f32),
        mesh=mesh,
        compiler_params=pltpu.CompilerParams(needs_layout_passes=False),
        scratch_types=[
            pltpu.VMEM((C3,), _i32),
            pltpu.VMEM((C3,), _i32),
            pltpu.VMEM((C3, HID), _f32),
            pltpu.VMEM((C3, HID), _f32),
            pltpu.VMEM((C3, BOND), _f32),
            pltpu.SemaphoreType.DMA,
        ],
    )(u, v, t, i, j)


def kernel(emb, edge_index, rbf, i, j, W_gat, att_src, att_dst, b_gat,
           W_rbf, b_rbf, W_lin, b_lin, W_out, b_out):
    src = edge_index[0]
    dst = edge_index[1]
    W1 = W_out[:HID]
    W2 = W_out[HID:2 * HID]
    W3 = W_out[2 * HID:]
    b_gat4 = b_gat.reshape(HEAD, HID)
    b_lin2 = b_lin.reshape(1, HID)
    b_rbf2 = b_rbf.reshape(1, HID)
    b_out2 = b_out.reshape(1, BOND)

    xw0, xw1, xw2, xw3, asN, adN = _k1(emb, W_gat, att_src, att_dst)
    xwall = jnp.concatenate([xw0, xw1, xw2, xw3], axis=0)
    asF = asN.reshape(-1)
    adF = adN.reshape(-1)
    denp, exmat = _s1(asF, adF, src, dst)
    denp = denp.reshape(NW, DRW * 128)[:, :N * HEAD]
    sp = _s2(xwall, exmat, src, dst)
    t = _k3(rbf, W_rbf, b_rbf2, W3, b_out2)
    x, u, v = _k2(sp, denp.reshape(NW, N, HEAD), emb, b_gat4, W_lin, b_lin2,
                  W1, W2)
    edge_out = _s3(u, v, t, i, j)
    return edge_out, x


# speedup vs baseline: 17.2062x; 1.3583x over previous
import jax
import jax.numpy as jnp
from jax import lax
from jax.experimental import pallas as pl
from jax.experimental.pallas import tpu as pltpu
from jax.experimental.pallas import tpu_sc as plsc

N = 10000
E = 320000
HID = 128
HEAD = 4
RAD = 16
BOND = 128

NC = 2
NS = 16
NW = NC * NS
ET = E // NW
RT = N // NS

_f32 = jnp.float32
_i32 = jnp.int32


BN1 = 400
G1 = N // BN1


def _k1_body(emb_ref, wg_ref, asv_ref, adv_ref,
             xw0_ref, xw1_ref, xw2_ref, xw3_ref, as_ref, ad_ref):
    xw = jnp.dot(emb_ref[...], wg_ref[...], preferred_element_type=_f32)
    asv = asv_ref[...]
    adv = adv_ref[...]
    xw_refs = (xw0_ref, xw1_ref, xw2_ref, xw3_ref)
    a_s = []
    a_d = []
    for h in range(HEAD):
        xh = xw[:, h * HID:(h + 1) * HID]
        xw_refs[h][...] = xh
        a_s.append(jnp.sum(xh * asv[h][None, :], axis=1)[:, None])
        a_d.append(jnp.sum(xh * adv[h][None, :], axis=1)[:, None])
    as_ref[...] = jnp.concatenate(a_s, axis=1)
    ad_ref[...] = jnp.concatenate(a_d, axis=1)


def _k1(emb, W_gat, att_src, att_dst):
    xw_spec = pl.BlockSpec((BN1, HID), lambda i: (i, 0))
    xw_shape = jax.ShapeDtypeStruct((N, HID), _f32)
    a_spec = pl.BlockSpec((BN1, HEAD), lambda i: (i, 0))
    a_shape = jax.ShapeDtypeStruct((N, HEAD), _f32)
    return pl.pallas_call(
        _k1_body,
        grid=(G1,),
        in_specs=[
            pl.BlockSpec((BN1, HID), lambda i: (i, 0)),
            pl.BlockSpec((HID, HEAD * HID), lambda i: (0, 0)),
            pl.BlockSpec((HEAD, HID), lambda i: (0, 0)),
            pl.BlockSpec((HEAD, HID), lambda i: (0, 0)),
        ],
        out_specs=[xw_spec, xw_spec, xw_spec, xw_spec, a_spec, a_spec],
        out_shape=[xw_shape, xw_shape, xw_shape, xw_shape, a_shape, a_shape],
    )(emb, W_gat, att_src, att_dst)


C1 = 80
NCH1 = ET // C1
DRW = 320


def _s1_body(as_hbm, ad_hbm, src_hbm, dst_hbm, denp_hbm, ex_hbm,
             as_v, ad_v, acc_v, srcv, dstv, exb):
    c = lax.axis_index("c")
    s = lax.axis_index("s")
    wid = c * NS + s
    base = wid * ET
    pltpu.sync_copy(as_hbm, as_v)
    pltpu.sync_copy(ad_hbm, ad_v)

    @pl.loop(0, DRW)
    def _zero(z):
        for g in range(8):
            acc_v[z, pl.ds(g * 16, 16)] = jnp.zeros((16,), _f32)

    @pl.loop(0, NCH1)
    def _chunk(ch):
        off = base + ch * C1
        pltpu.sync_copy(src_hbm.at[pl.ds(off, C1)], srcv)
        pltpu.sync_copy(dst_hbm.at[pl.ds(off, C1)], dstv)
        for g in range(C1 // 16):
            sv = srcv[pl.ds(g * 16, 16)]
            dv = dstv[pl.ds(g * 16, 16)]
            for h in range(HEAD):
                a = plsc.load_gather(as_v, [sv * HEAD + h])
                b = plsc.load_gather(ad_v, [dv * HEAD + h])
                al = a + b
                ex = jnp.exp(jnp.maximum(al, 0.2 * al))
                exb[h, pl.ds(g * 16, 16)] = ex
                fidx = dv * HEAD + h
                plsc.addupdate_scatter(acc_v, [fidx >> 7, fidx & 127], ex)
        for h in range(HEAD):
            pltpu.sync_copy(exb.at[h], ex_hbm.at[pl.ds(h * E + off, C1)])

    pltpu.sync_copy(acc_v, denp_hbm.at[wid])


def _s1(asT, adT, src, dst):
    mesh = plsc.VectorSubcoreMesh(core_axis_name="c", subcore_axis_name="s",
                                  num_cores=NC, num_subcores=NS)
    return pl.kernel(
        _s1_body,
        out_type=[jax.ShapeDtypeStruct((NW, DRW, 128), _f32),
                  jax.ShapeDtypeStruct((HEAD * E,), _f32)],
        mesh=mesh,
        compiler_params=pltpu.CompilerParams(needs_layout_passes=False),
        scratch_types=[
            pltpu.VMEM((N * HEAD,), _f32),
            pltpu.VMEM((N * HEAD,), _f32),
            pltpu.VMEM((DRW, 128), _f32),
            pltpu.VMEM((C1,), _i32),
            pltpu.VMEM((C1,), _i32),
            pltpu.VMEM((HEAD, C1), _f32),
        ],
    )(asT, adT, src, dst)


C2 = 80
NCH2 = ET // C2
SL = 624
TAIL = N - NS * SL


def _s2_body(xw_hbm, ex_hbm, src_hbm, dst_hbm, sp_hbm,
             rows_v, zbuf, srcv, dstv, idxv, exb, acc_sh, sem):
    c = lax.axis_index("c")
    s = lax.axis_index("s")
    base = (c * NS + s) * ET

    @pl.loop(0, TAIL)
    def _zb(z):
        for g in range(HID // 16):
            zbuf[z, pl.ds(g * 16, 16)] = jnp.zeros((16,), _f32)

    @pl.loop(0, HEAD)
    def _head(h):
        @pl.loop(0, SL // TAIL)
        def _zslice(z):
            pltpu.sync_copy(zbuf, acc_sh.at[pl.ds(s * SL + z * TAIL, TAIL)])

        @pl.when(s == 0)
        def _ztail():
            pltpu.sync_copy(zbuf, acc_sh.at[pl.ds(NS * SL, TAIL)])
        plsc.subcore_barrier()

        @pl.loop(0, NCH2)
        def _chunk(ch):
            off = base + ch * C2
            pltpu.sync_copy(src_hbm.at[pl.ds(off, C2)], srcv)
            pltpu.sync_copy(ex_hbm.at[pl.ds(h * E + off, C2)], exb)
            pltpu.sync_copy(dst_hbm.at[pl.ds(off, C2)], dstv)
            hN = h * N
            for g in range(C2 // 16):
                idxv[pl.ds(g * 16, 16)] = srcv[pl.ds(g * 16, 16)] + hN
            pltpu.async_copy(xw_hbm.at[idxv], rows_v, sem).wait()
            for gk in range(C2 // 16):
                mv = exb[pl.ds(gk * 16, 16)]
                for kk in range(16):
                    k = gk * 16 + kk
                    m = mv[kk]
                    for g in range(HID // 16):
                        sl = pl.ds(g * 16, 16)
                        rows_v[k, sl] = rows_v[k, sl] * m
            pltpu.sync_copy(rows_v, acc_sh.at[dstv], add=True)

        plsc.subcore_barrier()
        pltpu.sync_copy(acc_sh.at[pl.ds(s * SL, SL)],
                        sp_hbm.at[c, h, pl.ds(s * SL, SL)])

        @pl.when(s == 0)
        def _ctail():
            pltpu.sync_copy(acc_sh.at[pl.ds(NS * SL, TAIL)],
                            sp_hbm.at[c, h, pl.ds(NS * SL, TAIL)])
        plsc.subcore_barrier()


def _s2(xwall, exmat, src, dst):
    mesh = plsc.VectorSubcoreMesh(core_axis_name="c", subcore_axis_name="s",
                                  num_cores=NC, num_subcores=NS)
    return pl.kernel(
        _s2_body,
        out_type=jax.ShapeDtypeStruct((NC, HEAD, N, HID), _f32),
        mesh=mesh,
        compiler_params=pltpu.CompilerParams(needs_layout_passes=False),
        scratch_types=[
            pltpu.VMEM((C2, HID), _f32),
            pltpu.VMEM((TAIL, HID), _f32),
            pltpu.VMEM((C2,), _i32),
            pltpu.VMEM((C2,), _i32),
            pltpu.VMEM((C2,), _i32),
            pltpu.VMEM((C2,), _f32),
            pltpu.VMEM_SHARED((N, HID), _f32),
            pltpu.SemaphoreType.DMA,
        ],
    )(xwall, exmat, src, dst)


BN2 = 400
G2 = N // BN2


def _k2_body(sp_ref, dn_ref, emb_ref, bgat_ref, wlin_ref, blin_ref,
             w1_ref, w2_ref, x_ref, u_ref, v_ref):
    sp = sp_ref[...]
    den = jnp.sum(dn_ref[...], axis=0)
    emb = emb_ref[...]
    bgat = bgat_ref[...]
    yns = []
    for h in range(HEAD):
        sh = sp[0, h] + sp[1, h]
        agg = sh / (den[:, h:h + 1] + 1e-16)
        y = agg + bgat[h][None, :] + emb
        m = jnp.mean(y, axis=1, keepdims=True)
        yc = y - m
        var = jnp.mean(yc * yc, axis=1, keepdims=True)
        yns.append(yc * lax.rsqrt(var + 1e-5))
    xcat = jnp.concatenate(yns, axis=1)
    xx = jnp.dot(xcat, wlin_ref[...], preferred_element_type=_f32)
    xx = jnp.maximum(xx + blin_ref[...], 0.0)
    x_ref[...] = xx
    u_ref[...] = jnp.dot(xx, w1_ref[...], preferred_element_type=_f32)
    v_ref[...] = jnp.dot(xx, w2_ref[...], preferred_element_type=_f32)


def _k2(sp, denp3, emb, b_gat4, W_lin, b_lin2, W1, W2):
    return pl.pallas_call(
        _k2_body,
        grid=(G2,),
        in_specs=[
            pl.BlockSpec((NC, HEAD, BN2, HID), lambda i: (0, 0, i, 0)),
            pl.BlockSpec((NW, BN2, HEAD), lambda i: (0, i, 0)),
            pl.BlockSpec((BN2, HID), lambda i: (i, 0)),
            pl.BlockSpec((HEAD, HID), lambda i: (0, 0)),
            pl.BlockSpec((HEAD * HID, HID), lambda i: (0, 0)),
            pl.BlockSpec((1, HID), lambda i: (0, 0)),
            pl.BlockSpec((HID, HID), lambda i: (0, 0)),
            pl.BlockSpec((HID, HID), lambda i: (0, 0)),
        ],
        out_specs=[
            pl.BlockSpec((BN2, HID), lambda i: (i, 0)),
            pl.BlockSpec((BN2, HID), lambda i: (i, 0)),
            pl.BlockSpec((BN2, HID), lambda i: (i, 0)),
        ],
        out_shape=[
            jax.ShapeDtypeStruct((N, HID), _f32),
            jax.ShapeDtypeStruct((N, HID), _f32),
            jax.ShapeDtypeStruct((N, HID), _f32),
        ],
    )(sp, denp3, emb, b_gat4, W_lin, b_lin2, W1, W2)


BE3 = 2000
G3 = E // BE3


def _k3_body(rbf_ref, wr_ref, br_ref, w3_ref, bo_ref, t_ref):
    h1 = jnp.dot(rbf_ref[...], wr_ref[...], preferred_element_type=_f32)
    h1 = jnp.maximum(h1 + br_ref[...], 0.0)
    t_ref[...] = jnp.dot(h1, w3_ref[...], preferred_element_type=_f32) + bo_ref[...]


def _k3(rbf, W_rbf, b_rbf2, W3, b_out2):
    return pl.pallas_call(
        _k3_body,
        grid=(G3,),
        in_specs=[
            pl.BlockSpec((BE3, RAD), lambda i: (i, 0)),
            pl.BlockSpec((RAD, HID), lambda i: (0, 0)),
            pl.BlockSpec((1, HID), lambda i: (0, 0)),
            pl.BlockSpec((HID, BOND), lambda i: (0, 0)),
            pl.BlockSpec((1, BOND), lambda i: (0, 0)),
        ],
        out_specs=[pl.BlockSpec((BE3, BOND), lambda i: (i, 0))],
        out_shape=[jax.ShapeDtypeStruct((E, BOND), _f32)],
    )(rbf, W_rbf, b_rbf2, W3, b_out2)[0]


C3 = 80
NCH3 = ET // C3


def _s3_chunk_in(i_hbm, j_hbm, t_hbm, u_hbm, v_hbm, iv, jv, bu, bv, bt, gsem,
                 off):
    pltpu.sync_copy(i_hbm.at[pl.ds(off, C3)], iv)
    pltpu.sync_copy(j_hbm.at[pl.ds(off, C3)], jv)
    pltpu.async_copy(u_hbm.at[iv], bu, gsem)
    pltpu.async_copy(v_hbm.at[jv], bv, gsem)
    pltpu.async_copy(t_hbm.at[pl.ds(off, C3)], bt, gsem)


def _s3_drain_in(u_hbm, v_hbm, t_hbm, iv, jv, bu, bv, bt, gsem, off):
    pltpu.make_async_copy(u_hbm.at[iv], bu, gsem).wait()
    pltpu.make_async_copy(v_hbm.at[jv], bv, gsem).wait()
    pltpu.make_async_copy(t_hbm.at[pl.ds(off, C3)], bt, gsem).wait()


def _s3_combine(bu, bv, bt):
    @pl.loop(0, C3)
    def _row(k):
        for g in range(BOND // 16):
            sl = pl.ds(g * 16, 16)
            bu[k, sl] = jnp.maximum(bu[k, sl] + bv[k, sl] + bt[k, sl], 0.0)


def _s3_body(u_hbm, v_hbm, t_hbm, i_hbm, j_hbm, out_hbm,
             iv0, jv0, bu0, bv0, bt0, iv1, jv1, bu1, bv1, bt1,
             gsem0, gsem1, osem0, osem1):
    c = lax.axis_index("c")
    s = lax.axis_index("s")
    base = (c * NS + s) * ET
    iv = (iv0, iv1)
    jv = (jv0, jv1)
    bu = (bu0, bu1)
    bv = (bv0, bv1)
    bt = (bt0, bt1)
    gsem = (gsem0, gsem1)
    osem = (osem0, osem1)

    _s3_chunk_in(i_hbm, j_hbm, t_hbm, u_hbm, v_hbm, iv0, jv0, bu0, bv0, bt0,
                 gsem0, base)

    @pl.loop(0, NCH3 // 2)
    def _pair(t):
        for b in range(2):
            ch = t * 2 + b
            nb = 1 - b
            off = base + ch * C3
            noff = off + C3
            if b == 0:
                @pl.when(t > 0)
                def _dr():
                    pltpu.make_async_copy(
                        bu[nb], out_hbm.at[pl.ds(base, C3)], osem[nb]).wait()
            else:
                pltpu.make_async_copy(
                    bu[nb], out_hbm.at[pl.ds(base, C3)], osem[nb]).wait()

            @pl.when(ch + 1 < NCH3)
            def _pf():
                _s3_chunk_in(i_hbm, j_hbm, t_hbm, u_hbm, v_hbm, iv[nb],
                             jv[nb], bu[nb], bv[nb], bt[nb], gsem[nb], noff)
            _s3_drain_in(u_hbm, v_hbm, t_hbm, iv[b], jv[b], bu[b], bv[b],
                         bt[b], gsem[b], off)
            _s3_combine(bu[b], bv[b], bt[b])
            pltpu.async_copy(bu[b], out_hbm.at[pl.ds(off, C3)], osem[b])

    ch = NCH3 - 1
    off = base + ch * C3
    _s3_drain_in(u_hbm, v_hbm, t_hbm, iv0, jv0, bu0, bv0, bt0, gsem0, off)
    _s3_combine(bu0, bv0, bt0)
    pltpu.sync_copy(bu0, out_hbm.at[pl.ds(off, C3)])
    pltpu.make_async_copy(bu1, out_hbm.at[pl.ds(base, C3)], osem1).wait()


def _s3(u, v, t, i, j):
    mesh = plsc.VectorSubcoreMesh(core_axis_name="c", subcore_axis_name="s",
                                  num_cores=NC, num_subcores=NS)
    buf = lambda: pltpu.VMEM((C3, HID), _f32)
    idx = lambda: pltpu.VMEM((C3,), _i32)
    return pl.kernel(
        _s3_body,
        out_type=jax.ShapeDtypeStruct((E, BOND), _f32),
        mesh=mesh,
        compiler_params=pltpu.CompilerParams(needs_layout_passes=False),
        scratch_types=[
            idx(), idx(), buf(), buf(), buf(),
            idx(), idx(), buf(), buf(), buf(),
            pltpu.SemaphoreType.DMA, pltpu.SemaphoreType.DMA,
            pltpu.SemaphoreType.DMA, pltpu.SemaphoreType.DMA,
        ],
    )(u, v, t, i, j)


def kernel(emb, edge_index, rbf, i, j, W_gat, att_src, att_dst, b_gat,
           W_rbf, b_rbf, W_lin, b_lin, W_out, b_out):
    src = edge_index[0]
    dst = edge_index[1]
    W1 = W_out[:HID]
    W2 = W_out[HID:2 * HID]
    W3 = W_out[2 * HID:]
    b_gat4 = b_gat.reshape(HEAD, HID)
    b_lin2 = b_lin.reshape(1, HID)
    b_rbf2 = b_rbf.reshape(1, HID)
    b_out2 = b_out.reshape(1, BOND)

    xw0, xw1, xw2, xw3, asN, adN = _k1(emb, W_gat, att_src, att_dst)
    xwall = jnp.concatenate([xw0, xw1, xw2, xw3], axis=0)
    asF = asN.reshape(-1)
    adF = adN.reshape(-1)
    denp, exmat = _s1(asF, adF, src, dst)
    denp = denp.reshape(NW, DRW * 128)[:, :N * HEAD]
    sp = _s2(xwall, exmat, src, dst)
    t = _k3(rbf, W_rbf, b_rbf2, W3, b_out2)
    x, u, v = _k2(sp, denp.reshape(NW, N, HEAD), emb, b_gat4, W_lin, b_lin2,
                  W1, W2)
    edge_out = _s3(u, v, t, i, j)
    return edge_out, x


# speedup vs baseline: 24.8010x; 1.4414x over previous
import jax
import jax.numpy as jnp
from jax import lax
from jax.experimental import pallas as pl
from jax.experimental.pallas import tpu as pltpu
from jax.experimental.pallas import tpu_sc as plsc

N = 10000
E = 320000
HID = 128
HEAD = 4
RAD = 16
BOND = 128

NC = 2
NS = 16
NW = NC * NS
ET = E // NW
RT = N // NS

_f32 = jnp.float32
_i32 = jnp.int32


BN1 = 400
G1 = N // BN1


def _k1_body(emb_ref, wg_ref, asv_ref, adv_ref,
             xw0_ref, xw1_ref, xw2_ref, xw3_ref, as_ref, ad_ref):
    xw = jnp.dot(emb_ref[...], wg_ref[...], preferred_element_type=_f32)
    asv = asv_ref[...]
    adv = adv_ref[...]
    xw_refs = (xw0_ref, xw1_ref, xw2_ref, xw3_ref)
    a_s = []
    a_d = []
    for h in range(HEAD):
        xh = xw[:, h * HID:(h + 1) * HID]
        xw_refs[h][...] = xh
        a_s.append(jnp.sum(xh * asv[h][None, :], axis=1)[:, None])
        a_d.append(jnp.sum(xh * adv[h][None, :], axis=1)[:, None])
    as_ref[...] = jnp.concatenate(a_s, axis=1)
    ad_ref[...] = jnp.concatenate(a_d, axis=1)


def _k1(emb, W_gat, att_src, att_dst):
    xw_spec = pl.BlockSpec((BN1, HID), lambda i: (i, 0))
    xw_shape = jax.ShapeDtypeStruct((N, HID), _f32)
    a_spec = pl.BlockSpec((BN1, HEAD), lambda i: (i, 0))
    a_shape = jax.ShapeDtypeStruct((N, HEAD), _f32)
    return pl.pallas_call(
        _k1_body,
        grid=(G1,),
        in_specs=[
            pl.BlockSpec((BN1, HID), lambda i: (i, 0)),
            pl.BlockSpec((HID, HEAD * HID), lambda i: (0, 0)),
            pl.BlockSpec((HEAD, HID), lambda i: (0, 0)),
            pl.BlockSpec((HEAD, HID), lambda i: (0, 0)),
        ],
        out_specs=[xw_spec, xw_spec, xw_spec, xw_spec, a_spec, a_spec],
        out_shape=[xw_shape, xw_shape, xw_shape, xw_shape, a_shape, a_shape],
    )(emb, W_gat, att_src, att_dst)


C1 = 80
NCH1 = ET // C1
DRW = 320


def _s1_body(as_hbm, ad_hbm, src_hbm, dst_hbm, denp_hbm, ex_hbm,
             as_v, ad_v, acc_v, srcv, dstv, exb):
    c = lax.axis_index("c")
    s = lax.axis_index("s")
    wid = c * NS + s
    base = wid * ET
    pltpu.sync_copy(as_hbm, as_v)
    pltpu.sync_copy(ad_hbm, ad_v)

    @pl.loop(0, DRW)
    def _zero(z):
        for g in range(8):
            acc_v[z, pl.ds(g * 16, 16)] = jnp.zeros((16,), _f32)

    @pl.loop(0, NCH1)
    def _chunk(ch):
        off = base + ch * C1
        pltpu.sync_copy(src_hbm.at[pl.ds(off, C1)], srcv)
        pltpu.sync_copy(dst_hbm.at[pl.ds(off, C1)], dstv)
        for g in range(C1 // 16):
            sv = srcv[pl.ds(g * 16, 16)]
            dv = dstv[pl.ds(g * 16, 16)]
            for h in range(HEAD):
                a = plsc.load_gather(as_v, [sv * HEAD + h])
                b = plsc.load_gather(ad_v, [dv * HEAD + h])
                al = a + b
                ex = jnp.exp(jnp.maximum(al, 0.2 * al))
                exb[h, pl.ds(g * 16, 16)] = ex
                fidx = dv * HEAD + h
                plsc.addupdate_scatter(acc_v, [fidx >> 7, fidx & 127], ex)
        for h in range(HEAD):
            pltpu.sync_copy(exb.at[h], ex_hbm.at[pl.ds(h * E + off, C1)])

    pltpu.sync_copy(acc_v, denp_hbm.at[wid])


def _s1(asT, adT, src, dst):
    mesh = plsc.VectorSubcoreMesh(core_axis_name="c", subcore_axis_name="s",
                                  num_cores=NC, num_subcores=NS)
    return pl.kernel(
        _s1_body,
        out_type=[jax.ShapeDtypeStruct((NW, DRW, 128), _f32),
                  jax.ShapeDtypeStruct((HEAD * E,), _f32)],
        mesh=mesh,
        compiler_params=pltpu.CompilerParams(needs_layout_passes=False),
        scratch_types=[
            pltpu.VMEM((N * HEAD,), _f32),
            pltpu.VMEM((N * HEAD,), _f32),
            pltpu.VMEM((DRW, 128), _f32),
            pltpu.VMEM((C1,), _i32),
            pltpu.VMEM((C1,), _i32),
            pltpu.VMEM((HEAD, C1), _f32),
        ],
    )(asT, adT, src, dst)


C2 = 80
NCH2 = ET // C2
SL = 624
TAIL = N - NS * SL


def _s2_in(xw_hbm, ex_hbm, src_hbm, dst_hbm, srcv, dstv, idxv, exv, rows,
           gsem, off, hoff, hN):
    pltpu.sync_copy(src_hbm.at[pl.ds(off, C2)], srcv)
    pltpu.sync_copy(dst_hbm.at[pl.ds(off, C2)], dstv)
    for g in range(C2 // 16):
        idxv[pl.ds(g * 16, 16)] = srcv[pl.ds(g * 16, 16)] + hN
    pltpu.async_copy(ex_hbm.at[pl.ds(hoff, C2)], exv, gsem)
    pltpu.async_copy(xw_hbm.at[idxv], rows, gsem)


def _s2_drain_in(xw_hbm, ex_hbm, idxv, exv, rows, gsem, hoff):
    pltpu.make_async_copy(ex_hbm.at[pl.ds(hoff, C2)], exv, gsem).wait()
    pltpu.make_async_copy(xw_hbm.at[idxv], rows, gsem).wait()


def _s2_scale(exv, rows):
    @pl.loop(0, C2 // 16)
    def _gk(gk):
        mv = exv[pl.ds(gk * 16, 16)]
        for kk in range(16):
            m = mv[kk]
            for g in range(HID // 16):
                sl = pl.ds(g * 16, 16)
                rows[gk * 16 + kk, sl] = rows[gk * 16 + kk, sl] * m


def _s2_body(xw_hbm, ex_hbm, src_hbm, dst_hbm, sp_hbm,
             rows0, srcv0, dstv0, idxv0, exv0,
             rows1, srcv1, dstv1, idxv1, exv1,
             zbuf, gsem0, gsem1, ssem0, ssem1, acc_sh):
    c = lax.axis_index("c")
    s = lax.axis_index("s")
    base = (c * NS + s) * ET
    rows = (rows0, rows1)
    srcv = (srcv0, srcv1)
    dstv = (dstv0, dstv1)
    idxv = (idxv0, idxv1)
    exv = (exv0, exv1)
    gsem = (gsem0, gsem1)
    ssem = (ssem0, ssem1)

    @pl.loop(0, TAIL)
    def _zb(z):
        for g in range(HID // 16):
            zbuf[z, pl.ds(g * 16, 16)] = jnp.zeros((16,), _f32)

    @pl.loop(0, HEAD)
    def _head(h):
        hN = h * N
        hE = h * E

        @pl.loop(0, SL // TAIL)
        def _zslice(z):
            pltpu.sync_copy(zbuf, acc_sh.at[pl.ds(s * SL + z * TAIL, TAIL)])

        @pl.when(s == 0)
        def _ztail():
            pltpu.sync_copy(zbuf, acc_sh.at[pl.ds(NS * SL, TAIL)])
        plsc.subcore_barrier()

        _s2_in(xw_hbm, ex_hbm, src_hbm, dst_hbm, srcv0, dstv0, idxv0, exv0,
               rows0, gsem0, base, hE + base, hN)

        @pl.loop(0, NCH2 // 2)
        def _pair(t):
            for b in range(2):
                ch = t * 2 + b
                nb = 1 - b
                off = base + ch * C2
                if b == 0:
                    @pl.when(t > 0)
                    def _dr():
                        pltpu.make_async_copy(
                            rows[nb], acc_sh.at[dstv[nb]], ssem[nb]).wait()
                else:
                    pltpu.make_async_copy(
                        rows[nb], acc_sh.at[dstv[nb]], ssem[nb]).wait()

                @pl.when(ch + 1 < NCH2)
                def _pf():
                    _s2_in(xw_hbm, ex_hbm, src_hbm, dst_hbm, srcv[nb],
                           dstv[nb], idxv[nb], exv[nb], rows[nb], gsem[nb],
                           off + C2, hE + off + C2, hN)
                _s2_drain_in(xw_hbm, ex_hbm, idxv[b], exv[b], rows[b],
                             gsem[b], hE + off)
                _s2_scale(exv[b], rows[b])
                pltpu.async_copy(rows[b], acc_sh.at[dstv[b]], ssem[b],
                                 add=True)

        off = base + (NCH2 - 1) * C2
        _s2_drain_in(xw_hbm, ex_hbm, idxv0, exv0, rows0, gsem0, hE + off)
        _s2_scale(exv0, rows0)
        pltpu.sync_copy(rows0, acc_sh.at[dstv0], add=True)
        pltpu.make_async_copy(rows1, acc_sh.at[dstv1], ssem1).wait()

        plsc.subcore_barrier()
        pltpu.sync_copy(acc_sh.at[pl.ds(s * SL, SL)],
                        sp_hbm.at[c, h, pl.ds(s * SL, SL)])

        @pl.when(s == 0)
        def _ctail():
            pltpu.sync_copy(acc_sh.at[pl.ds(NS * SL, TAIL)],
                            sp_hbm.at[c, h, pl.ds(NS * SL, TAIL)])
        plsc.subcore_barrier()


def _s2(xwall, exmat, src, dst):
    mesh = plsc.VectorSubcoreMesh(core_axis_name="c", subcore_axis_name="s",
                                  num_cores=NC, num_subcores=NS)
    return pl.kernel(
        _s2_body,
        out_type=jax.ShapeDtypeStruct((NC, HEAD, N, HID), _f32),
        mesh=mesh,
        compiler_params=pltpu.CompilerParams(needs_layout_passes=False),
        scratch_types=[
            pltpu.VMEM((C2, HID), _f32),
            pltpu.VMEM((C2,), _i32),
            pltpu.VMEM((C2,), _i32),
            pltpu.VMEM((C2,), _i32),
            pltpu.VMEM((C2,), _f32),
            pltpu.VMEM((C2, HID), _f32),
            pltpu.VMEM((C2,), _i32),
            pltpu.VMEM((C2,), _i32),
            pltpu.VMEM((C2,), _i32),
            pltpu.VMEM((C2,), _f32),
            pltpu.VMEM((TAIL, HID), _f32),
            pltpu.SemaphoreType.DMA, pltpu.SemaphoreType.DMA,
            pltpu.SemaphoreType.DMA, pltpu.SemaphoreType.DMA,
            pltpu.VMEM_SHARED((N, HID), _f32),
        ],
    )(xwall, exmat, src, dst)


BN2 = 400
G2 = N // BN2


def _k2_body(sp_ref, dn_ref, emb_ref, bgat_ref, wlin_ref, blin_ref,
             w1_ref, w2_ref, x_ref, u_ref, v_ref):
    sp = sp_ref[...]
    den = jnp.sum(dn_ref[...], axis=0)
    emb = emb_ref[...]
    bgat = bgat_ref[...]
    yns = []
    for h in range(HEAD):
        sh = sp[0, h] + sp[1, h]
        agg = sh / (den[:, h:h + 1] + 1e-16)
        y = agg + bgat[h][None, :] + emb
        m = jnp.mean(y, axis=1, keepdims=True)
        yc = y - m
        var = jnp.mean(yc * yc, axis=1, keepdims=True)
        yns.append(yc * lax.rsqrt(var + 1e-5))
    xcat = jnp.concatenate(yns, axis=1)
    xx = jnp.dot(xcat, wlin_ref[...], preferred_element_type=_f32)
    xx = jnp.maximum(xx + blin_ref[...], 0.0)
    x_ref[...] = xx
    u_ref[...] = jnp.dot(xx, w1_ref[...], preferred_element_type=_f32)
    v_ref[...] = jnp.dot(xx, w2_ref[...], preferred_element_type=_f32)


def _k2(sp, denp3, emb, b_gat4, W_lin, b_lin2, W1, W2):
    return pl.pallas_call(
        _k2_body,
        grid=(G2,),
        in_specs=[
            pl.BlockSpec((NC, HEAD, BN2, HID), lambda i: (0, 0, i, 0)),
            pl.BlockSpec((NW, BN2, HEAD), lambda i: (0, i, 0)),
            pl.BlockSpec((BN2, HID), lambda i: (i, 0)),
            pl.BlockSpec((HEAD, HID), lambda i: (0, 0)),
            pl.BlockSpec((HEAD * HID, HID), lambda i: (0, 0)),
            pl.BlockSpec((1, HID), lambda i: (0, 0)),
            pl.BlockSpec((HID, HID), lambda i: (0, 0)),
            pl.BlockSpec((HID, HID), lambda i: (0, 0)),
        ],
        out_specs=[
            pl.BlockSpec((BN2, HID), lambda i: (i, 0)),
            pl.BlockSpec((BN2, HID), lambda i: (i, 0)),
            pl.BlockSpec((BN2, HID), lambda i: (i, 0)),
        ],
        out_shape=[
            jax.ShapeDtypeStruct((N, HID), _f32),
            jax.ShapeDtypeStruct((N, HID), _f32),
            jax.ShapeDtypeStruct((N, HID), _f32),
        ],
    )(sp, denp3, emb, b_gat4, W_lin, b_lin2, W1, W2)


BE3 = 2000
G3 = E // BE3


def _k3_body(rbf_ref, wr_ref, br_ref, w3_ref, bo_ref, t_ref):
    h1 = jnp.dot(rbf_ref[...], wr_ref[...], preferred_element_type=_f32)
    h1 = jnp.maximum(h1 + br_ref[...], 0.0)
    t_ref[...] = jnp.dot(h1, w3_ref[...], preferred_element_type=_f32) + bo_ref[...]


def _k3(rbf, W_rbf, b_rbf2, W3, b_out2):
    return pl.pallas_call(
        _k3_body,
        grid=(G3,),
        in_specs=[
            pl.BlockSpec((BE3, RAD), lambda i: (i, 0)),
            pl.BlockSpec((RAD, HID), lambda i: (0, 0)),
            pl.BlockSpec((1, HID), lambda i: (0, 0)),
            pl.BlockSpec((HID, BOND), lambda i: (0, 0)),
            pl.BlockSpec((1, BOND), lambda i: (0, 0)),
        ],
        out_specs=[pl.BlockSpec((BE3, BOND), lambda i: (i, 0))],
        out_shape=[jax.ShapeDtypeStruct((E, BOND), _f32)],
    )(rbf, W_rbf, b_rbf2, W3, b_out2)[0]


C3 = 80
NCH3 = ET // C3


def _s3_chunk_in(i_hbm, j_hbm, t_hbm, u_hbm, v_hbm, iv, jv, bu, bv, bt, gsem,
                 off):
    pltpu.sync_copy(i_hbm.at[pl.ds(off, C3)], iv)
    pltpu.sync_copy(j_hbm.at[pl.ds(off, C3)], jv)
    pltpu.async_copy(u_hbm.at[iv], bu, gsem)
    pltpu.async_copy(v_hbm.at[jv], bv, gsem)
    pltpu.async_copy(t_hbm.at[pl.ds(off, C3)], bt, gsem)


def _s3_drain_in(u_hbm, v_hbm, t_hbm, iv, jv, bu, bv, bt, gsem, off):
    pltpu.make_async_copy(u_hbm.at[iv], bu, gsem).wait()
    pltpu.make_async_copy(v_hbm.at[jv], bv, gsem).wait()
    pltpu.make_async_copy(t_hbm.at[pl.ds(off, C3)], bt, gsem).wait()


def _s3_combine(bu, bv, bt):
    @pl.loop(0, C3)
    def _row(k):
        for g in range(BOND // 16):
            sl = pl.ds(g * 16, 16)
            bu[k, sl] = jnp.maximum(bu[k, sl] + bv[k, sl] + bt[k, sl], 0.0)


def _s3_body(u_hbm, v_hbm, t_hbm, i_hbm, j_hbm, out_hbm,
             iv0, jv0, bu0, bv0, bt0, iv1, jv1, bu1, bv1, bt1,
             gsem0, gsem1, osem0, osem1):
    c = lax.axis_index("c")
    s = lax.axis_index("s")
    base = (c * NS + s) * ET
    iv = (iv0, iv1)
    jv = (jv0, jv1)
    bu = (bu0, bu1)
    bv = (bv0, bv1)
    bt = (bt0, bt1)
    gsem = (gsem0, gsem1)
    osem = (osem0, osem1)

    _s3_chunk_in(i_hbm, j_hbm, t_hbm, u_hbm, v_hbm, iv0, jv0, bu0, bv0, bt0,
                 gsem0, base)

    @pl.loop(0, NCH3 // 2)
    def _pair(t):
        for b in range(2):
            ch = t * 2 + b
            nb = 1 - b
            off = base + ch * C3
            noff = off + C3
            if b == 0:
                @pl.when(t > 0)
                def _dr():
                    pltpu.make_async_copy(
                        bu[nb], out_hbm.at[pl.ds(base, C3)], osem[nb]).wait()
            else:
                pltpu.make_async_copy(
                    bu[nb], out_hbm.at[pl.ds(base, C3)], osem[nb]).wait()

            @pl.when(ch + 1 < NCH3)
            def _pf():
                _s3_chunk_in(i_hbm, j_hbm, t_hbm, u_hbm, v_hbm, iv[nb],
                             jv[nb], bu[nb], bv[nb], bt[nb], gsem[nb], noff)
            _s3_drain_in(u_hbm, v_hbm, t_hbm, iv[b], jv[b], bu[b], bv[b],
                         bt[b], gsem[b], off)
            _s3_combine(bu[b], bv[b], bt[b])
            pltpu.async_copy(bu[b], out_hbm.at[pl.ds(off, C3)], osem[b])

    ch = NCH3 - 1
    off = base + ch * C3
    _s3_drain_in(u_hbm, v_hbm, t_hbm, iv0, jv0, bu0, bv0, bt0, gsem0, off)
    _s3_combine(bu0, bv0, bt0)
    pltpu.sync_copy(bu0, out_hbm.at[pl.ds(off, C3)])
    pltpu.make_async_copy(bu1, out_hbm.at[pl.ds(base, C3)], osem1).wait()


def _s3(u, v, t, i, j):
    mesh = plsc.VectorSubcoreMesh(core_axis_name="c", subcore_axis_name="s",
                                  num_cores=NC, num_subcores=NS)
    buf = lambda: pltpu.VMEM((C3, HID), _f32)
    idx = lambda: pltpu.VMEM((C3,), _i32)
    return pl.kernel(
        _s3_body,
        out_type=jax.ShapeDtypeStruct((E, BOND), _f32),
        mesh=mesh,
        compiler_params=pltpu.CompilerParams(needs_layout_passes=False),
        scratch_types=[
            idx(), idx(), buf(), buf(), buf(),
            idx(), idx(), buf(), buf(), buf(),
            pltpu.SemaphoreType.DMA, pltpu.SemaphoreType.DMA,
            pltpu.SemaphoreType.DMA, pltpu.SemaphoreType.DMA,
        ],
    )(u, v, t, i, j)


def kernel(emb, edge_index, rbf, i, j, W_gat, att_src, att_dst, b_gat,
           W_rbf, b_rbf, W_lin, b_lin, W_out, b_out):
    src = edge_index[0]
    dst = edge_index[1]
    W1 = W_out[:HID]
    W2 = W_out[HID:2 * HID]
    W3 = W_out[2 * HID:]
    b_gat4 = b_gat.reshape(HEAD, HID)
    b_lin2 = b_lin.reshape(1, HID)
    b_rbf2 = b_rbf.reshape(1, HID)
    b_out2 = b_out.reshape(1, BOND)

    xw0, xw1, xw2, xw3, asN, adN = _k1(emb, W_gat, att_src, att_dst)
    xwall = jnp.concatenate([xw0, xw1, xw2, xw3], axis=0)
    asF = asN.reshape(-1)
    adF = adN.reshape(-1)
    denp, exmat = _s1(asF, adF, src, dst)
    denp = denp.reshape(NW, DRW * 128)[:, :N * HEAD]
    sp = _s2(xwall, exmat, src, dst)
    t = _k3(rbf, W_rbf, b_rbf2, W3, b_out2)
    x, u, v = _k2(sp, denp.reshape(NW, N, HEAD), emb, b_gat4, W_lin, b_lin2,
                  W1, W2)
    edge_out = _s3(u, v, t, i, j)
    return edge_out, x


# speedup vs baseline: 29.2718x; 1.1803x over previous
import jax
import jax.numpy as jnp
from jax import lax
from jax.experimental import pallas as pl
from jax.experimental.pallas import tpu as pltpu
from jax.experimental.pallas import tpu_sc as plsc

N = 10000
E = 320000
HID = 128
HEAD = 4
RAD = 16
BOND = 128

NC = 2
NS = 16
NW = NC * NS
ET = E // NW
RT = N // NS

_f32 = jnp.float32
_i32 = jnp.int32


BN1 = 400
G1 = N // BN1


def _k1_body(emb_ref, wg_ref, asv_ref, adv_ref,
             xw0_ref, xw1_ref, xw2_ref, xw3_ref, as_ref, ad_ref):
    xw = jnp.dot(emb_ref[...], wg_ref[...], preferred_element_type=_f32)
    asv = asv_ref[...]
    adv = adv_ref[...]
    xw_refs = (xw0_ref, xw1_ref, xw2_ref, xw3_ref)
    a_s = []
    a_d = []
    for h in range(HEAD):
        xh = xw[:, h * HID:(h + 1) * HID]
        xw_refs[h][...] = xh
        a_s.append(jnp.sum(xh * asv[h][None, :], axis=1)[:, None])
        a_d.append(jnp.sum(xh * adv[h][None, :], axis=1)[:, None])
    as_ref[...] = jnp.concatenate(a_s, axis=1)
    ad_ref[...] = jnp.concatenate(a_d, axis=1)


def _k1(emb, W_gat, att_src, att_dst):
    xw_spec = pl.BlockSpec((BN1, HID), lambda i: (i, 0))
    xw_shape = jax.ShapeDtypeStruct((N, HID), _f32)
    a_spec = pl.BlockSpec((BN1, HEAD), lambda i: (i, 0))
    a_shape = jax.ShapeDtypeStruct((N, HEAD), _f32)
    return pl.pallas_call(
        _k1_body,
        grid=(G1,),
        in_specs=[
            pl.BlockSpec((BN1, HID), lambda i: (i, 0)),
            pl.BlockSpec((HID, HEAD * HID), lambda i: (0, 0)),
            pl.BlockSpec((HEAD, HID), lambda i: (0, 0)),
            pl.BlockSpec((HEAD, HID), lambda i: (0, 0)),
        ],
        out_specs=[xw_spec, xw_spec, xw_spec, xw_spec, a_spec, a_spec],
        out_shape=[xw_shape, xw_shape, xw_shape, xw_shape, a_shape, a_shape],
    )(emb, W_gat, att_src, att_dst)


C1 = 80
NCH1 = ET // C1
DRW = 320


def _s1_body(as_hbm, ad_hbm, src_hbm, dst_hbm, denp_hbm, ex_hbm,
             as_v, ad_v, acc_v, srcv, dstv, exb):
    c = lax.axis_index("c")
    s = lax.axis_index("s")
    wid = c * NS + s
    base = wid * ET
    pltpu.sync_copy(as_hbm, as_v)
    pltpu.sync_copy(ad_hbm, ad_v)

    @pl.loop(0, DRW)
    def _zero(z):
        for g in range(8):
            acc_v[z, pl.ds(g * 16, 16)] = jnp.zeros((16,), _f32)

    @pl.loop(0, NCH1)
    def _chunk(ch):
        off = base + ch * C1
        pltpu.sync_copy(src_hbm.at[pl.ds(off, C1)], srcv)
        pltpu.sync_copy(dst_hbm.at[pl.ds(off, C1)], dstv)
        for g in range(C1 // 16):
            sv = srcv[pl.ds(g * 16, 16)]
            dv = dstv[pl.ds(g * 16, 16)]
            for h in range(HEAD):
                a = plsc.load_gather(as_v, [sv * HEAD + h])
                b = plsc.load_gather(ad_v, [dv * HEAD + h])
                al = a + b
                ex = jnp.exp(jnp.maximum(al, 0.2 * al))
                exb[h, pl.ds(g * 16, 16)] = ex
                fidx = dv * HEAD + h
                plsc.addupdate_scatter(acc_v, [fidx >> 7, fidx & 127], ex)
        for h in range(HEAD):
            pltpu.sync_copy(exb.at[h], ex_hbm.at[pl.ds(h * E + off, C1)])

    pltpu.sync_copy(acc_v, denp_hbm.at[wid])


def _s1(asT, adT, src, dst):
    mesh = plsc.VectorSubcoreMesh(core_axis_name="c", subcore_axis_name="s",
                                  num_cores=NC, num_subcores=NS)
    return pl.kernel(
        _s1_body,
        out_type=[jax.ShapeDtypeStruct((NW, DRW, 128), _f32),
                  jax.ShapeDtypeStruct((HEAD * E,), _f32)],
        mesh=mesh,
        compiler_params=pltpu.CompilerParams(needs_layout_passes=False),
        scratch_types=[
            pltpu.VMEM((N * HEAD,), _f32),
            pltpu.VMEM((N * HEAD,), _f32),
            pltpu.VMEM((DRW, 128), _f32),
            pltpu.VMEM((C1,), _i32),
            pltpu.VMEM((C1,), _i32),
            pltpu.VMEM((HEAD, C1), _f32),
        ],
    )(asT, adT, src, dst)


C2 = 128
NCH2F = (ET // C2)
C2T = ET - NCH2F * C2
SL = 624
TAIL = N - NS * SL


def _s2_in(xw_hbm, ex_hbm, src_hbm, dst_hbm, srcv, dstv, exv, rows,
           gsem, off, hoff, hN):
    pltpu.sync_copy(src_hbm.at[pl.ds(off, C2)], srcv)
    pltpu.sync_copy(dst_hbm.at[pl.ds(off, C2)], dstv)
    for g in range(C2 // 16):
        srcv[pl.ds(g * 16, 16)] = srcv[pl.ds(g * 16, 16)] + hN
    pltpu.async_copy(ex_hbm.at[pl.ds(hoff, C2)], exv, gsem)
    pltpu.async_copy(xw_hbm.at[srcv], rows, gsem)


def _s2_drain_in(xw_hbm, ex_hbm, srcv, exv, rows, gsem, hoff):
    pltpu.make_async_copy(ex_hbm.at[pl.ds(hoff, C2)], exv, gsem).wait()
    pltpu.make_async_copy(xw_hbm.at[srcv], rows, gsem).wait()


def _s2_scale(exv, rows):
    @pl.loop(0, C2 // 16)
    def _gk(gk):
        mv = exv[pl.ds(gk * 16, 16)]
        for kk in range(16):
            m = mv[kk]
            for g in range(HID // 16):
                sl = pl.ds(g * 16, 16)
                rows[gk * 16 + kk, sl] = rows[gk * 16 + kk, sl] * m


def _s2_body(xw_hbm, ex_hbm, src_hbm, dst_hbm, zro_hbm, sp_hbm,
             rows0, srcv0, dstv0, exv0,
             rows1, srcv1, dstv1, exv1,
             rows2, srcv2, dstv2, exv2,
             srcvt, dstvt,
             gsem0, gsem1, gsem2, ssem0, ssem1, ssem2, tsem, acc_sh):
    c = lax.axis_index("c")
    s = lax.axis_index("s")
    base = (c * NS + s) * ET
    rows = (rows0, rows1, rows2)
    srcv = (srcv0, srcv1, srcv2)
    dstv = (dstv0, dstv1, dstv2)
    exv = (exv0, exv1, exv2)
    gsem = (gsem0, gsem1, gsem2)
    ssem = (ssem0, ssem1, ssem2)

    @pl.loop(0, HEAD)
    def _head(h):
        hN = h * N
        hE = h * E

        pltpu.sync_copy(zro_hbm, acc_sh.at[pl.ds(s * SL, SL)])

        @pl.when(s == 0)
        def _ztail():
            pltpu.sync_copy(zro_hbm.at[pl.ds(0, TAIL)],
                            acc_sh.at[pl.ds(NS * SL, TAIL)])
        plsc.subcore_barrier()

        _s2_in(xw_hbm, ex_hbm, src_hbm, dst_hbm, srcv0, dstv0, exv0,
               rows0, gsem0, base, hE + base, hN)
        _s2_in(xw_hbm, ex_hbm, src_hbm, dst_hbm, srcv1, dstv1, exv1,
               rows1, gsem1, base + C2, hE + base + C2, hN)

        @pl.loop(0, NCH2F // 3)
        def _trip(t):
            for b in range(3):
                ch = t * 3 + b
                nb = (b + 2) % 3
                off = base + ch * C2
                _s2_drain_in(xw_hbm, ex_hbm, srcv[b], exv[b], rows[b],
                             gsem[b], hE + off)
                _s2_scale(exv[b], rows[b])
                if b == 0:
                    @pl.when(t > 0)
                    def _dr():
                        pltpu.make_async_copy(
                            rows[nb], acc_sh.at[dstv[nb]], ssem[nb]).wait()
                else:
                    pltpu.make_async_copy(
                        rows[nb], acc_sh.at[dstv[nb]], ssem[nb]).wait()

                @pl.when(ch + 2 < NCH2F)
                def _pf():
                    _s2_in(xw_hbm, ex_hbm, src_hbm, dst_hbm, srcv[nb],
                           dstv[nb], exv[nb], rows[nb], gsem[nb],
                           off + 2 * C2, hE + off + 2 * C2, hN)
                pltpu.async_copy(rows[b], acc_sh.at[dstv[b]], ssem[b],
                                 add=True)

        toff = base + NCH2F * C2
        pltpu.sync_copy(src_hbm.at[pl.ds(toff, C2T)], srcvt)
        pltpu.sync_copy(dst_hbm.at[pl.ds(toff, C2T)], dstvt)
        srcvt[...] = srcvt[...] + hN
        pltpu.async_copy(xw_hbm.at[srcvt], rows0.at[pl.ds(0, C2T)], tsem)
        pltpu.sync_copy(ex_hbm.at[pl.ds(hE + toff, C2T)],
                        exv0.at[pl.ds(0, C2T)])
        pltpu.make_async_copy(
            xw_hbm.at[srcvt], rows0.at[pl.ds(0, C2T)], tsem).wait()
        mv = exv0[pl.ds(0, C2T)]
        for kk in range(C2T):
            m = mv[kk]
            for g in range(HID // 16):
                sl = pl.ds(g * 16, 16)
                rows0[kk, sl] = rows0[kk, sl] * m
        pltpu.make_async_copy(rows2, acc_sh.at[dstv2], ssem2).wait()
        pltpu.sync_copy(rows0.at[pl.ds(0, C2T)], acc_sh.at[dstvt], add=True)

        plsc.subcore_barrier()
        pltpu.sync_copy(acc_sh.at[pl.ds(s * SL, SL)],
                        sp_hbm.at[c, h, pl.ds(s * SL, SL)])

        @pl.when(s == 0)
        def _ctail():
            pltpu.sync_copy(acc_sh.at[pl.ds(NS * SL, TAIL)],
                            sp_hbm.at[c, h, pl.ds(NS * SL, TAIL)])
        plsc.subcore_barrier()


def _s2(xwall, exmat, src, dst, zro):
    mesh = plsc.VectorSubcoreMesh(core_axis_name="c", subcore_axis_name="s",
                                  num_cores=NC, num_subcores=NS)
    return pl.kernel(
        _s2_body,
        out_type=jax.ShapeDtypeStruct((NC, HEAD, N, HID), _f32),
        mesh=mesh,
        compiler_params=pltpu.CompilerParams(needs_layout_passes=False),
        scratch_types=[
            pltpu.VMEM((C2, HID), _f32),
            pltpu.VMEM((C2,), _i32),
            pltpu.VMEM((C2,), _i32),
            pltpu.VMEM((C2,), _f32),
            pltpu.VMEM((C2, HID), _f32),
            pltpu.VMEM((C2,), _i32),
            pltpu.VMEM((C2,), _i32),
            pltpu.VMEM((C2,), _f32),
            pltpu.VMEM((C2, HID), _f32),
            pltpu.VMEM((C2,), _i32),
            pltpu.VMEM((C2,), _i32),
            pltpu.VMEM((C2,), _f32),
            pltpu.VMEM((C2T,), _i32),
            pltpu.VMEM((C2T,), _i32),
            pltpu.SemaphoreType.DMA, pltpu.SemaphoreType.DMA,
            pltpu.SemaphoreType.DMA, pltpu.SemaphoreType.DMA,
            pltpu.SemaphoreType.DMA, pltpu.SemaphoreType.DMA,
            pltpu.SemaphoreType.DMA,
            pltpu.VMEM_SHARED((N, HID), _f32),
        ],
    )(xwall, exmat, src, dst, zro)


BN2 = 400
G2 = N // BN2


def _k2_body(sp_ref, dn_ref, emb_ref, bgat_ref, wlin_ref, blin_ref,
             w1_ref, w2_ref, x_ref, u_ref, v_ref):
    sp = sp_ref[...]
    den = jnp.sum(dn_ref[...], axis=0)
    emb = emb_ref[...]
    bgat = bgat_ref[...]
    yns = []
    for h in range(HEAD):
        sh = sp[0, h] + sp[1, h]
        agg = sh / (den[:, h:h + 1] + 1e-16)
        y = agg + bgat[h][None, :] + emb
        m = jnp.mean(y, axis=1, keepdims=True)
        yc = y - m
        var = jnp.mean(yc * yc, axis=1, keepdims=True)
        yns.append(yc * lax.rsqrt(var + 1e-5))
    xcat = jnp.concatenate(yns, axis=1)
    xx = jnp.dot(xcat, wlin_ref[...], preferred_element_type=_f32)
    xx = jnp.maximum(xx + blin_ref[...], 0.0)
    x_ref[...] = xx
    u_ref[...] = jnp.dot(xx, w1_ref[...], preferred_element_type=_f32)
    v_ref[...] = jnp.dot(xx, w2_ref[...], preferred_element_type=_f32)


def _k2(sp, denp3, emb, b_gat4, W_lin, b_lin2, W1, W2):
    return pl.pallas_call(
        _k2_body,
        grid=(G2,),
        in_specs=[
            pl.BlockSpec((NC, HEAD, BN2, HID), lambda i: (0, 0, i, 0)),
            pl.BlockSpec((NW, BN2, HEAD), lambda i: (0, i, 0)),
            pl.BlockSpec((BN2, HID), lambda i: (i, 0)),
            pl.BlockSpec((HEAD, HID), lambda i: (0, 0)),
            pl.BlockSpec((HEAD * HID, HID), lambda i: (0, 0)),
            pl.BlockSpec((1, HID), lambda i: (0, 0)),
            pl.BlockSpec((HID, HID), lambda i: (0, 0)),
            pl.BlockSpec((HID, HID), lambda i: (0, 0)),
        ],
        out_specs=[
            pl.BlockSpec((BN2, HID), lambda i: (i, 0)),
            pl.BlockSpec((BN2, HID), lambda i: (i, 0)),
            pl.BlockSpec((BN2, HID), lambda i: (i, 0)),
        ],
        out_shape=[
            jax.ShapeDtypeStruct((N, HID), _f32),
            jax.ShapeDtypeStruct((N, HID), _f32),
            jax.ShapeDtypeStruct((N, HID), _f32),
        ],
    )(sp, denp3, emb, b_gat4, W_lin, b_lin2, W1, W2)


BE3 = 2000
G3 = E // BE3


def _k3_body(rbf_ref, wr_ref, br_ref, w3_ref, bo_ref, t_ref):
    h1 = jnp.dot(rbf_ref[...], wr_ref[...], preferred_element_type=_f32)
    h1 = jnp.maximum(h1 + br_ref[...], 0.0)
    t_ref[...] = jnp.dot(h1, w3_ref[...], preferred_element_type=_f32) + bo_ref[...]


def _k3(rbf, W_rbf, b_rbf2, W3, b_out2):
    return pl.pallas_call(
        _k3_body,
        grid=(G3,),
        in_specs=[
            pl.BlockSpec((BE3, RAD), lambda i: (i, 0)),
            pl.BlockSpec((RAD, HID), lambda i: (0, 0)),
            pl.BlockSpec((1, HID), lambda i: (0, 0)),
            pl.BlockSpec((HID, BOND), lambda i: (0, 0)),
            pl.BlockSpec((1, BOND), lambda i: (0, 0)),
        ],
        out_specs=[pl.BlockSpec((BE3, BOND), lambda i: (i, 0))],
        out_shape=[jax.ShapeDtypeStruct((E, BOND), _f32)],
    )(rbf, W_rbf, b_rbf2, W3, b_out2)[0]


C3 = 80
NCH3 = ET // C3


def _s3_chunk_in(i_hbm, j_hbm, t_hbm, u_hbm, v_hbm, iv, jv, bu, bv, bt, gsem,
                 off):
    pltpu.sync_copy(i_hbm.at[pl.ds(off, C3)], iv)
    pltpu.sync_copy(j_hbm.at[pl.ds(off, C3)], jv)
    pltpu.async_copy(u_hbm.at[iv], bu, gsem)
    pltpu.async_copy(v_hbm.at[jv], bv, gsem)
    pltpu.async_copy(t_hbm.at[pl.ds(off, C3)], bt, gsem)


def _s3_drain_in(u_hbm, v_hbm, t_hbm, iv, jv, bu, bv, bt, gsem, off):
    pltpu.make_async_copy(u_hbm.at[iv], bu, gsem).wait()
    pltpu.make_async_copy(v_hbm.at[jv], bv, gsem).wait()
    pltpu.make_async_copy(t_hbm.at[pl.ds(off, C3)], bt, gsem).wait()


def _s3_combine(bu, bv, bt):
    @pl.loop(0, C3)
    def _row(k):
        for g in range(BOND // 16):
            sl = pl.ds(g * 16, 16)
            bu[k, sl] = jnp.maximum(bu[k, sl] + bv[k, sl] + bt[k, sl], 0.0)


def _s3_body(u_hbm, v_hbm, t_hbm, i_hbm, j_hbm, out_hbm,
             iv0, jv0, bu0, bv0, bt0, iv1, jv1, bu1, bv1, bt1,
             gsem0, gsem1, osem0, osem1):
    c = lax.axis_index("c")
    s = lax.axis_index("s")
    base = (c * NS + s) * ET
    iv = (iv0, iv1)
    jv = (jv0, jv1)
    bu = (bu0, bu1)
    bv = (bv0, bv1)
    bt = (bt0, bt1)
    gsem = (gsem0, gsem1)
    osem = (osem0, osem1)

    _s3_chunk_in(i_hbm, j_hbm, t_hbm, u_hbm, v_hbm, iv0, jv0, bu0, bv0, bt0,
                 gsem0, base)

    @pl.loop(0, NCH3 // 2)
    def _pair(t):
        for b in range(2):
            ch = t * 2 + b
            nb = 1 - b
            off = base + ch * C3
            noff = off + C3
            if b == 0:
                @pl.when(t > 0)
                def _dr():
                    pltpu.make_async_copy(
                        bu[nb], out_hbm.at[pl.ds(base, C3)], osem[nb]).wait()
            else:
                pltpu.make_async_copy(
                    bu[nb], out_hbm.at[pl.ds(base, C3)], osem[nb]).wait()

            @pl.when(ch + 1 < NCH3)
            def _pf():
                _s3_chunk_in(i_hbm, j_hbm, t_hbm, u_hbm, v_hbm, iv[nb],
                             jv[nb], bu[nb], bv[nb], bt[nb], gsem[nb], noff)
            _s3_drain_in(u_hbm, v_hbm, t_hbm, iv[b], jv[b], bu[b], bv[b],
                         bt[b], gsem[b], off)
            _s3_combine(bu[b], bv[b], bt[b])
            pltpu.async_copy(bu[b], out_hbm.at[pl.ds(off, C3)], osem[b])

    ch = NCH3 - 1
    off = base + ch * C3
    _s3_drain_in(u_hbm, v_hbm, t_hbm, iv0, jv0, bu0, bv0, bt0, gsem0, off)
    _s3_combine(bu0, bv0, bt0)
    pltpu.sync_copy(bu0, out_hbm.at[pl.ds(off, C3)])
    pltpu.make_async_copy(bu1, out_hbm.at[pl.ds(base, C3)], osem1).wait()


def _s3(u, v, t, i, j):
    mesh = plsc.VectorSubcoreMesh(core_axis_name="c", subcore_axis_name="s",
                                  num_cores=NC, num_subcores=NS)
    buf = lambda: pltpu.VMEM((C3, HID), _f32)
    idx = lambda: pltpu.VMEM((C3,), _i32)
    return pl.kernel(
        _s3_body,
        out_type=jax.ShapeDtypeStruct((E, BOND), _f32),
        mesh=mesh,
        compiler_params=pltpu.CompilerParams(needs_layout_passes=False),
        scratch_types=[
            idx(), idx(), buf(), buf(), buf(),
            idx(), idx(), buf(), buf(), buf(),
            pltpu.SemaphoreType.DMA, pltpu.SemaphoreType.DMA,
            pltpu.SemaphoreType.DMA, pltpu.SemaphoreType.DMA,
        ],
    )(u, v, t, i, j)


def kernel(emb, edge_index, rbf, i, j, W_gat, att_src, att_dst, b_gat,
           W_rbf, b_rbf, W_lin, b_lin, W_out, b_out):
    src = edge_index[0]
    dst = edge_index[1]
    W1 = W_out[:HID]
    W2 = W_out[HID:2 * HID]
    W3 = W_out[2 * HID:]
    b_gat4 = b_gat.reshape(HEAD, HID)
    b_lin2 = b_lin.reshape(1, HID)
    b_rbf2 = b_rbf.reshape(1, HID)
    b_out2 = b_out.reshape(1, BOND)

    xw0, xw1, xw2, xw3, asN, adN = _k1(emb, W_gat, att_src, att_dst)
    xwall = jnp.concatenate([xw0, xw1, xw2, xw3], axis=0)
    asF = asN.reshape(-1)
    adF = adN.reshape(-1)
    denp, exmat = _s1(asF, adF, src, dst)
    denp = denp.reshape(NW, DRW * 128)[:, :N * HEAD]
    zro = jnp.zeros((SL, HID), _f32)
    sp = _s2(xwall, exmat, src, dst, zro)
    t = _k3(rbf, W_rbf, b_rbf2, W3, b_out2)
    x, u, v = _k2(sp, denp.reshape(NW, N, HEAD), emb, b_gat4, W_lin, b_lin2,
                  W1, W2)
    edge_out = _s3(u, v, t, i, j)
    return edge_out, x


# speedup vs baseline: 32.2651x; 1.1023x over previous
import jax
import jax.numpy as jnp
from jax import lax
from jax.experimental import pallas as pl
from jax.experimental.pallas import tpu as pltpu
from jax.experimental.pallas import tpu_sc as plsc

N = 10000
E = 320000
HID = 128
HEAD = 4
RAD = 16
BOND = 128

NC = 2
NS = 16
NW = NC * NS
ET = E // NW
RT = N // NS

_f32 = jnp.float32
_i32 = jnp.int32


BN1 = 400
G1 = N // BN1


def _k1_body(emb_ref, wg_ref, asv_ref, adv_ref,
             xw0_ref, xw1_ref, xw2_ref, xw3_ref, as_ref, ad_ref):
    xw = jnp.dot(emb_ref[...], wg_ref[...], preferred_element_type=_f32)
    asv = asv_ref[...]
    adv = adv_ref[...]
    xw_refs = (xw0_ref, xw1_ref, xw2_ref, xw3_ref)
    a_s = []
    a_d = []
    for h in range(HEAD):
        xh = xw[:, h * HID:(h + 1) * HID]
        xw_refs[h][...] = xh
        a_s.append(jnp.sum(xh * asv[h][None, :], axis=1)[:, None])
        a_d.append(jnp.sum(xh * adv[h][None, :], axis=1)[:, None])
    as_ref[...] = jnp.concatenate(a_s, axis=1)
    ad_ref[...] = jnp.concatenate(a_d, axis=1)


def _k1(emb, W_gat, att_src, att_dst):
    xw_spec = pl.BlockSpec((BN1, HID), lambda i: (i, 0))
    xw_shape = jax.ShapeDtypeStruct((N, HID), _f32)
    a_spec = pl.BlockSpec((BN1, HEAD), lambda i: (i, 0))
    a_shape = jax.ShapeDtypeStruct((N, HEAD), _f32)
    return pl.pallas_call(
        _k1_body,
        grid=(G1,),
        in_specs=[
            pl.BlockSpec((BN1, HID), lambda i: (i, 0)),
            pl.BlockSpec((HID, HEAD * HID), lambda i: (0, 0)),
            pl.BlockSpec((HEAD, HID), lambda i: (0, 0)),
            pl.BlockSpec((HEAD, HID), lambda i: (0, 0)),
        ],
        out_specs=[xw_spec, xw_spec, xw_spec, xw_spec, a_spec, a_spec],
        out_shape=[xw_shape, xw_shape, xw_shape, xw_shape, a_shape, a_shape],
    )(emb, W_gat, att_src, att_dst)


C1 = 80
NCH1 = ET // C1
DRW = 320


def _s1_idx_in(src_hbm, dst_hbm, srcv, dstv, isem, off):
    pltpu.async_copy(src_hbm.at[pl.ds(off, C1)], srcv, isem)
    pltpu.async_copy(dst_hbm.at[pl.ds(off, C1)], dstv, isem)


def _s1_idx_drain(src_hbm, dst_hbm, srcv, dstv, isem, off):
    pltpu.make_async_copy(src_hbm.at[pl.ds(off, C1)], srcv, isem).wait()
    pltpu.make_async_copy(dst_hbm.at[pl.ds(off, C1)], dstv, isem).wait()


def _s1_compute(as_v, ad_v, acc_v, srcv, dstv, exb):
    for g in range(C1 // 16):
        sv = srcv[pl.ds(g * 16, 16)]
        dv = dstv[pl.ds(g * 16, 16)]
        for h in range(HEAD):
            a = plsc.load_gather(as_v, [sv * HEAD + h])
            b = plsc.load_gather(ad_v, [dv * HEAD + h])
            al = a + b
            ex = jnp.exp(jnp.maximum(al, 0.2 * al))
            exb[h, pl.ds(g * 16, 16)] = ex
            fidx = dv * HEAD + h
            plsc.addupdate_scatter(acc_v, [fidx >> 7, fidx & 127], ex)


def _s1_ex_out(ex_hbm, exb, wsem, off, fire):
    for h in range(HEAD):
        cp = pltpu.make_async_copy(exb.at[h], ex_hbm.at[pl.ds(h * E + off, C1)],
                                   wsem)
        if fire:
            cp.start()
        else:
            cp.wait()


def _s1_body(as_hbm, ad_hbm, src_hbm, dst_hbm, denp_hbm, ex_hbm,
             as_v, ad_v, acc_v, srcv0, dstv0, exb0, srcv1, dstv1, exb1,
             isem0, isem1, wsem0, wsem1):
    c = lax.axis_index("c")
    s = lax.axis_index("s")
    wid = c * NS + s
    base = wid * ET
    srcv = (srcv0, srcv1)
    dstv = (dstv0, dstv1)
    exb = (exb0, exb1)
    isem = (isem0, isem1)
    wsem = (wsem0, wsem1)
    pltpu.sync_copy(as_hbm, as_v)
    pltpu.sync_copy(ad_hbm, ad_v)

    @pl.loop(0, DRW)
    def _zero(z):
        for g in range(8):
            acc_v[z, pl.ds(g * 16, 16)] = jnp.zeros((16,), _f32)

    _s1_idx_in(src_hbm, dst_hbm, srcv0, dstv0, isem0, base)

    @pl.loop(0, (NCH1 - 1) // 2)
    def _pair(t):
        for b in range(2):
            ch = t * 2 + b
            nb = 1 - b
            off = base + ch * C1
            _s1_idx_drain(src_hbm, dst_hbm, srcv[b], dstv[b], isem[b], off)
            _s1_idx_in(src_hbm, dst_hbm, srcv[nb], dstv[nb], isem[nb],
                       off + C1)
            @pl.when(t > 0)
            def _dr():
                _s1_ex_out(ex_hbm, exb[b], wsem[b], off, False)
            _s1_compute(as_v, ad_v, acc_v, srcv[b], dstv[b], exb[b])
            _s1_ex_out(ex_hbm, exb[b], wsem[b], off, True)

    off = base + (NCH1 - 1) * C1
    _s1_idx_drain(src_hbm, dst_hbm, srcv0, dstv0, isem0, off)
    _s1_ex_out(ex_hbm, exb0, wsem0, off, False)
    _s1_compute(as_v, ad_v, acc_v, srcv0, dstv0, exb0)
    for h in range(HEAD):
        pltpu.sync_copy(exb0.at[h], ex_hbm.at[pl.ds(h * E + off, C1)])
    _s1_ex_out(ex_hbm, exb1, wsem1, off, False)

    pltpu.sync_copy(acc_v, denp_hbm.at[wid])


def _s1(asT, adT, src, dst):
    mesh = plsc.VectorSubcoreMesh(core_axis_name="c", subcore_axis_name="s",
                                  num_cores=NC, num_subcores=NS)
    return pl.kernel(
        _s1_body,
        out_type=[jax.ShapeDtypeStruct((NW, DRW, 128), _f32),
                  jax.ShapeDtypeStruct((HEAD * E,), _f32)],
        mesh=mesh,
        compiler_params=pltpu.CompilerParams(needs_layout_passes=False),
        scratch_types=[
            pltpu.VMEM((N * HEAD,), _f32),
            pltpu.VMEM((N * HEAD,), _f32),
            pltpu.VMEM((DRW, 128), _f32),
            pltpu.VMEM((C1,), _i32),
            pltpu.VMEM((C1,), _i32),
            pltpu.VMEM((HEAD, C1), _f32),
            pltpu.VMEM((C1,), _i32),
            pltpu.VMEM((C1,), _i32),
            pltpu.VMEM((HEAD, C1), _f32),
            pltpu.SemaphoreType.DMA, pltpu.SemaphoreType.DMA,
            pltpu.SemaphoreType.DMA, pltpu.SemaphoreType.DMA,
        ],
    )(asT, adT, src, dst)


C2 = 128
NCH2F = (ET // C2)
C2T = ET - NCH2F * C2
SL = 624
TAIL = N - NS * SL


def _s2_in(xw_hbm, ex_hbm, src_hbm, dst_hbm, srcv, dstv, exv, rows,
           gsem, off, hoff, hN):
    pltpu.sync_copy(src_hbm.at[pl.ds(off, C2)], srcv)
    pltpu.sync_copy(dst_hbm.at[pl.ds(off, C2)], dstv)
    for g in range(C2 // 16):
        srcv[pl.ds(g * 16, 16)] = srcv[pl.ds(g * 16, 16)] + hN
    pltpu.async_copy(ex_hbm.at[pl.ds(hoff, C2)], exv, gsem)
    pltpu.async_copy(xw_hbm.at[srcv], rows, gsem)


def _s2_drain_in(xw_hbm, ex_hbm, srcv, exv, rows, gsem, hoff):
    pltpu.make_async_copy(ex_hbm.at[pl.ds(hoff, C2)], exv, gsem).wait()
    pltpu.make_async_copy(xw_hbm.at[srcv], rows, gsem).wait()


def _s2_scale(exv, rows):
    @pl.loop(0, C2 // 16)
    def _gk(gk):
        mv = exv[pl.ds(gk * 16, 16)]
        for kk in range(16):
            m = mv[kk]
            for g in range(HID // 16):
                sl = pl.ds(g * 16, 16)
                rows[gk * 16 + kk, sl] = rows[gk * 16 + kk, sl] * m


def _s2_body(xw_hbm, ex_hbm, src_hbm, dst_hbm, zro_hbm, sp_hbm,
             rows0, srcv0, dstv0, exv0,
             rows1, srcv1, dstv1, exv1,
             rows2, srcv2, dstv2, exv2,
             srcvt, dstvt,
             gsem0, gsem1, gsem2, ssem0, ssem1, ssem2, tsem, acc_sh):
    c = lax.axis_index("c")
    s = lax.axis_index("s")
    base = (c * NS + s) * ET
    rows = (rows0, rows1, rows2)
    srcv = (srcv0, srcv1, srcv2)
    dstv = (dstv0, dstv1, dstv2)
    exv = (exv0, exv1, exv2)
    gsem = (gsem0, gsem1, gsem2)
    ssem = (ssem0, ssem1, ssem2)

    @pl.loop(0, HEAD)
    def _head(h):
        hN = h * N
        hE = h * E

        pltpu.sync_copy(zro_hbm, acc_sh.at[pl.ds(s * SL, SL)])

        @pl.when(s == 0)
        def _ztail():
            pltpu.sync_copy(zro_hbm.at[pl.ds(0, TAIL)],
                            acc_sh.at[pl.ds(NS * SL, TAIL)])
        plsc.subcore_barrier()

        _s2_in(xw_hbm, ex_hbm, src_hbm, dst_hbm, srcv0, dstv0, exv0,
               rows0, gsem0, base, hE + base, hN)
        _s2_in(xw_hbm, ex_hbm, src_hbm, dst_hbm, srcv1, dstv1, exv1,
               rows1, gsem1, base + C2, hE + base + C2, hN)

        @pl.loop(0, NCH2F // 3)
        def _trip(t):
            for b in range(3):
                ch = t * 3 + b
                nb = (b + 2) % 3
                off = base + ch * C2
                _s2_drain_in(xw_hbm, ex_hbm, srcv[b], exv[b], rows[b],
                             gsem[b], hE + off)
                _s2_scale(exv[b], rows[b])
                if b == 0:
                    @pl.when(t > 0)
                    def _dr():
                        pltpu.make_async_copy(
                            rows[nb], acc_sh.at[dstv[nb]], ssem[nb]).wait()
                else:
                    pltpu.make_async_copy(
                        rows[nb], acc_sh.at[dstv[nb]], ssem[nb]).wait()

                @pl.when(ch + 2 < NCH2F)
                def _pf():
                    _s2_in(xw_hbm, ex_hbm, src_hbm, dst_hbm, srcv[nb],
                           dstv[nb], exv[nb], rows[nb], gsem[nb],
                           off + 2 * C2, hE + off + 2 * C2, hN)
                pltpu.async_copy(rows[b], acc_sh.at[dstv[b]], ssem[b],
                                 add=True)

        toff = base + NCH2F * C2
        pltpu.sync_copy(src_hbm.at[pl.ds(toff, C2T)], srcvt)
        pltpu.sync_copy(dst_hbm.at[pl.ds(toff, C2T)], dstvt)
        srcvt[...] = srcvt[...] + hN
        pltpu.async_copy(xw_hbm.at[srcvt], rows0.at[pl.ds(0, C2T)], tsem)
        pltpu.sync_copy(ex_hbm.at[pl.ds(hE + toff, C2T)],
                        exv0.at[pl.ds(0, C2T)])
        pltpu.make_async_copy(
            xw_hbm.at[srcvt], rows0.at[pl.ds(0, C2T)], tsem).wait()
        mv = exv0[pl.ds(0, C2T)]
        for kk in range(C2T):
            m = mv[kk]
            for g in range(HID // 16):
                sl = pl.ds(g * 16, 16)
                rows0[kk, sl] = rows0[kk, sl] * m
        pltpu.make_async_copy(rows2, acc_sh.at[dstv2], ssem2).wait()
        pltpu.sync_copy(rows0.at[pl.ds(0, C2T)], acc_sh.at[dstvt], add=True)

        plsc.subcore_barrier()
        pltpu.sync_copy(acc_sh.at[pl.ds(s * SL, SL)],
                        sp_hbm.at[c, h, pl.ds(s * SL, SL)])

        @pl.when(s == 0)
        def _ctail():
            pltpu.sync_copy(acc_sh.at[pl.ds(NS * SL, TAIL)],
                            sp_hbm.at[c, h, pl.ds(NS * SL, TAIL)])
        plsc.subcore_barrier()


def _s2(xwall, exmat, src, dst, zro):
    mesh = plsc.VectorSubcoreMesh(core_axis_name="c", subcore_axis_name="s",
                                  num_cores=NC, num_subcores=NS)
    return pl.kernel(
        _s2_body,
        out_type=jax.ShapeDtypeStruct((NC, HEAD, N, HID), _f32),
        mesh=mesh,
        compiler_params=pltpu.CompilerParams(needs_layout_passes=False),
        scratch_types=[
            pltpu.VMEM((C2, HID), _f32),
            pltpu.VMEM((C2,), _i32),
            pltpu.VMEM((C2,), _i32),
            pltpu.VMEM((C2,), _f32),
            pltpu.VMEM((C2, HID), _f32),
            pltpu.VMEM((C2,), _i32),
            pltpu.VMEM((C2,), _i32),
            pltpu.VMEM((C2,), _f32),
            pltpu.VMEM((C2, HID), _f32),
            pltpu.VMEM((C2,), _i32),
            pltpu.VMEM((C2,), _i32),
            pltpu.VMEM((C2,), _f32),
            pltpu.VMEM((C2T,), _i32),
            pltpu.VMEM((C2T,), _i32),
            pltpu.SemaphoreType.DMA, pltpu.SemaphoreType.DMA,
            pltpu.SemaphoreType.DMA, pltpu.SemaphoreType.DMA,
            pltpu.SemaphoreType.DMA, pltpu.SemaphoreType.DMA,
            pltpu.SemaphoreType.DMA,
            pltpu.VMEM_SHARED((N, HID), _f32),
        ],
    )(xwall, exmat, src, dst, zro)


BN2 = 400
G2 = N // BN2


def _k2_body(sp_ref, dn_ref, emb_ref, bgat_ref, wlin_ref, blin_ref,
             w1_ref, w2_ref, x_ref, u_ref, v_ref):
    sp = sp_ref[...]
    den = jnp.sum(dn_ref[...], axis=0)
    emb = emb_ref[...]
    bgat = bgat_ref[...]
    yns = []
    for h in range(HEAD):
        sh = sp[0, h] + sp[1, h]
        agg = sh / (den[:, h:h + 1] + 1e-16)
        y = agg + bgat[h][None, :] + emb
        m = jnp.mean(y, axis=1, keepdims=True)
        yc = y - m
        var = jnp.mean(yc * yc, axis=1, keepdims=True)
        yns.append(yc * lax.rsqrt(var + 1e-5))
    xcat = jnp.concatenate(yns, axis=1)
    xx = jnp.dot(xcat, wlin_ref[...], preferred_element_type=_f32)
    xx = jnp.maximum(xx + blin_ref[...], 0.0)
    x_ref[...] = xx
    u_ref[...] = jnp.dot(xx, w1_ref[...], preferred_element_type=_f32)
    v_ref[...] = jnp.dot(xx, w2_ref[...], preferred_element_type=_f32)


def _k2(sp, denp3, emb, b_gat4, W_lin, b_lin2, W1, W2):
    return pl.pallas_call(
        _k2_body,
        grid=(G2,),
        in_specs=[
            pl.BlockSpec((NC, HEAD, BN2, HID), lambda i: (0, 0, i, 0)),
            pl.BlockSpec((NW, BN2, HEAD), lambda i: (0, i, 0)),
            pl.BlockSpec((BN2, HID), lambda i: (i, 0)),
            pl.BlockSpec((HEAD, HID), lambda i: (0, 0)),
            pl.BlockSpec((HEAD * HID, HID), lambda i: (0, 0)),
            pl.BlockSpec((1, HID), lambda i: (0, 0)),
            pl.BlockSpec((HID, HID), lambda i: (0, 0)),
            pl.BlockSpec((HID, HID), lambda i: (0, 0)),
        ],
        out_specs=[
            pl.BlockSpec((BN2, HID), lambda i: (i, 0)),
            pl.BlockSpec((BN2, HID), lambda i: (i, 0)),
            pl.BlockSpec((BN2, HID), lambda i: (i, 0)),
        ],
        out_shape=[
            jax.ShapeDtypeStruct((N, HID), _f32),
            jax.ShapeDtypeStruct((N, HID), _f32),
            jax.ShapeDtypeStruct((N, HID), _f32),
        ],
    )(sp, denp3, emb, b_gat4, W_lin, b_lin2, W1, W2)


BE3 = 2000
G3 = E // BE3


def _k3_body(rbf_ref, wr_ref, br_ref, w3_ref, bo_ref, t_ref):
    h1 = jnp.dot(rbf_ref[...], wr_ref[...], preferred_element_type=_f32)
    h1 = jnp.maximum(h1 + br_ref[...], 0.0)
    t_ref[...] = jnp.dot(h1, w3_ref[...], preferred_element_type=_f32) + bo_ref[...]


def _k3(rbf, W_rbf, b_rbf2, W3, b_out2):
    return pl.pallas_call(
        _k3_body,
        grid=(G3,),
        in_specs=[
            pl.BlockSpec((BE3, RAD), lambda i: (i, 0)),
            pl.BlockSpec((RAD, HID), lambda i: (0, 0)),
            pl.BlockSpec((1, HID), lambda i: (0, 0)),
            pl.BlockSpec((HID, BOND), lambda i: (0, 0)),
            pl.BlockSpec((1, BOND), lambda i: (0, 0)),
        ],
        out_specs=[pl.BlockSpec((BE3, BOND), lambda i: (i, 0))],
        out_shape=[jax.ShapeDtypeStruct((E, BOND), _f32)],
    )(rbf, W_rbf, b_rbf2, W3, b_out2)[0]


C3 = 80
NCH3 = ET // C3


def _s3_chunk_in(i_hbm, j_hbm, t_hbm, u_hbm, v_hbm, iv, jv, bu, bv, bt, gsem,
                 off):
    pltpu.sync_copy(i_hbm.at[pl.ds(off, C3)], iv)
    pltpu.sync_copy(j_hbm.at[pl.ds(off, C3)], jv)
    pltpu.async_copy(u_hbm.at[iv], bu, gsem)
    pltpu.async_copy(v_hbm.at[jv], bv, gsem)
    pltpu.async_copy(t_hbm.at[pl.ds(off, C3)], bt, gsem)


def _s3_drain_in(u_hbm, v_hbm, t_hbm, iv, jv, bu, bv, bt, gsem, off):
    pltpu.make_async_copy(u_hbm.at[iv], bu, gsem).wait()
    pltpu.make_async_copy(v_hbm.at[jv], bv, gsem).wait()
    pltpu.make_async_copy(t_hbm.at[pl.ds(off, C3)], bt, gsem).wait()


def _s3_combine(bu, bv, bt):
    @pl.loop(0, C3)
    def _row(k):
        for g in range(BOND // 16):
            sl = pl.ds(g * 16, 16)
            bu[k, sl] = jnp.maximum(bu[k, sl] + bv[k, sl] + bt[k, sl], 0.0)


def _s3_body(u_hbm, v_hbm, t_hbm, i_hbm, j_hbm, out_hbm,
             iv0, jv0, bu0, bv0, bt0, iv1, jv1, bu1, bv1, bt1,
             gsem0, gsem1, osem0, osem1):
    c = lax.axis_index("c")
    s = lax.axis_index("s")
    base = (c * NS + s) * ET
    iv = (iv0, iv1)
    jv = (jv0, jv1)
    bu = (bu0, bu1)
    bv = (bv0, bv1)
    bt = (bt0, bt1)
    gsem = (gsem0, gsem1)
    osem = (osem0, osem1)

    _s3_chunk_in(i_hbm, j_hbm, t_hbm, u_hbm, v_hbm, iv0, jv0, bu0, bv0, bt0,
                 gsem0, base)

    @pl.loop(0, NCH3 // 2)
    def _pair(t):
        for b in range(2):
            ch = t * 2 + b
            nb = 1 - b
            off = base + ch * C3
            noff = off + C3
            if b == 0:
                @pl.when(t > 0)
                def _dr():
                    pltpu.make_async_copy(
                        bu[nb], out_hbm.at[pl.ds(base, C3)], osem[nb]).wait()
            else:
                pltpu.make_async_copy(
                    bu[nb], out_hbm.at[pl.ds(base, C3)], osem[nb]).wait()

            @pl.when(ch + 1 < NCH3)
            def _pf():
                _s3_chunk_in(i_hbm, j_hbm, t_hbm, u_hbm, v_hbm, iv[nb],
                             jv[nb], bu[nb], bv[nb], bt[nb], gsem[nb], noff)
            _s3_drain_in(u_hbm, v_hbm, t_hbm, iv[b], jv[b], bu[b], bv[b],
                         bt[b], gsem[b], off)
            _s3_combine(bu[b], bv[b], bt[b])
            pltpu.async_copy(bu[b], out_hbm.at[pl.ds(off, C3)], osem[b])

    ch = NCH3 - 1
    off = base + ch * C3
    _s3_drain_in(u_hbm, v_hbm, t_hbm, iv0, jv0, bu0, bv0, bt0, gsem0, off)
    _s3_combine(bu0, bv0, bt0)
    pltpu.sync_copy(bu0, out_hbm.at[pl.ds(off, C3)])
    pltpu.make_async_copy(bu1, out_hbm.at[pl.ds(base, C3)], osem1).wait()


def _s3(u, v, t, i, j):
    mesh = plsc.VectorSubcoreMesh(core_axis_name="c", subcore_axis_name="s",
                                  num_cores=NC, num_subcores=NS)
    buf = lambda: pltpu.VMEM((C3, HID), _f32)
    idx = lambda: pltpu.VMEM((C3,), _i32)
    return pl.kernel(
        _s3_body,
        out_type=jax.ShapeDtypeStruct((E, BOND), _f32),
        mesh=mesh,
        compiler_params=pltpu.CompilerParams(needs_layout_passes=False),
        scratch_types=[
            idx(), idx(), buf(), buf(), buf(),
            idx(), idx(), buf(), buf(), buf(),
            pltpu.SemaphoreType.DMA, pltpu.SemaphoreType.DMA,
            pltpu.SemaphoreType.DMA, pltpu.SemaphoreType.DMA,
        ],
    )(u, v, t, i, j)


def kernel(emb, edge_index, rbf, i, j, W_gat, att_src, att_dst, b_gat,
           W_rbf, b_rbf, W_lin, b_lin, W_out, b_out):
    src = edge_index[0]
    dst = edge_index[1]
    W1 = W_out[:HID]
    W2 = W_out[HID:2 * HID]
    W3 = W_out[2 * HID:]
    b_gat4 = b_gat.reshape(HEAD, HID)
    b_lin2 = b_lin.reshape(1, HID)
    b_rbf2 = b_rbf.reshape(1, HID)
    b_out2 = b_out.reshape(1, BOND)

    xw0, xw1, xw2, xw3, asN, adN = _k1(emb, W_gat, att_src, att_dst)
    xwall = jnp.concatenate([xw0, xw1, xw2, xw3], axis=0)
    asF = asN.reshape(-1)
    adF = adN.reshape(-1)
    denp, exmat = _s1(asF, adF, src, dst)
    denp = denp.reshape(NW, DRW * 128)[:, :N * HEAD]
    zro = jnp.zeros((SL, HID), _f32)
    sp = _s2(xwall, exmat, src, dst, zro)
    t = _k3(rbf, W_rbf, b_rbf2, W3, b_out2)
    x, u, v = _k2(sp, denp.reshape(NW, N, HEAD), emb, b_gat4, W_lin, b_lin2,
                  W1, W2)
    edge_out = _s3(u, v, t, i, j)
    return edge_out, x
